# Initial kernel scaffold; baseline (speedup 1.0000x reference)
import jax
import jax.numpy as jnp
from jax.experimental import pallas as pl


def kernel(x, edge_index, edge_attr, u, batch, params):
    raise NotImplementedError("write your pallas kernel here")


# speedup vs baseline: 1.5694x; 1.5694x over previous
import functools
import math

import jax
import jax.numpy as jnp
from jax import lax
from jax.experimental import pallas as pl
from jax.experimental.pallas import tpu as pltpu
from jax.experimental.pallas import tpu_sc as plsc

F32 = jnp.float32


def _prep_body(x_ref, wab_ref, xa_ref, xb_ref):
    y = jnp.dot(x_ref[...], wab_ref[...], preferred_element_type=F32)
    h = y.shape[1] // 2
    xa_ref[...] = y[:, :h]
    xb_ref[...] = y[:, h:]


def _node_projections(x, wa, wb, block=400):
    n, f = x.shape
    hid = wa.shape[1]
    wab = jnp.concatenate([wa, wb], axis=1)
    grid = (n // block,)
    return pl.pallas_call(
        _prep_body,
        grid=grid,
        in_specs=[
            pl.BlockSpec((block, f), lambda i: (i, 0)),
            pl.BlockSpec((f, 2 * hid), lambda i: (0, 0)),
        ],
        out_specs=[
            pl.BlockSpec((block, hid), lambda i: (i, 0)),
            pl.BlockSpec((block, hid), lambda i: (i, 0)),
        ],
        out_shape=[
            jax.ShapeDtypeStruct((n, hid), F32),
            jax.ShapeDtypeStruct((n, hid), F32),
        ],
    )(x, wab)


def _matmul_body(x_ref, w_ref, out_ref):
    out_ref[...] = jnp.dot(x_ref[...], w_ref[...], preferred_element_type=F32)


def _matmul(x, w, block=512):
    r, k = x.shape
    f = w.shape[1]
    return pl.pallas_call(
        _matmul_body,
        grid=(r // block,),
        in_specs=[
            pl.BlockSpec((block, k), lambda i: (i, 0)),
            pl.BlockSpec((k, f), lambda i: (0, 0)),
        ],
        out_specs=pl.BlockSpec((block, f), lambda i: (i, 0)),
        out_shape=jax.ShapeDtypeStruct((r, f), F32),
    )(x, w)


def _edge0_body(ga_ref, gb_ref, gc_ref, b1_ref, w2_ref, b2_ref, out_ref):
    h = jnp.maximum(ga_ref[...] + gb_ref[...] + gc_ref[...] + b1_ref[...], 0.0)
    out_ref[...] = jnp.dot(h, w2_ref[...], preferred_element_type=F32) + b2_ref[...]


def _edge_body(ga_ref, gb_ref, es_ref, wc_ref, b1_ref, w2_ref, b2_ref, out_ref):
    es = es_ref[...]
    ec = jnp.dot(es, wc_ref[...], preferred_element_type=F32)
    h = jnp.maximum(ga_ref[...] + gb_ref[...] + ec + b1_ref[...], 0.0)
    out_ref[...] = jnp.dot(h, w2_ref[...], preferred_element_type=F32) + b2_ref[...] + es


def _edge_mlp_l0(ga, gb, gc, b1, w2, b2, block=512):
    e, hid = ga.shape
    grid = (e // block,)
    vec = lambda v: v.reshape(1, -1)
    return pl.pallas_call(
        _edge0_body,
        grid=grid,
        in_specs=[
            pl.BlockSpec((block, hid), lambda i: (i, 0)),
            pl.BlockSpec((block, hid), lambda i: (i, 0)),
            pl.BlockSpec((block, hid), lambda i: (i, 0)),
            pl.BlockSpec((1, hid), lambda i: (0, 0)),
            pl.BlockSpec((hid, hid), lambda i: (0, 0)),
            pl.BlockSpec((1, hid), lambda i: (0, 0)),
        ],
        out_specs=pl.BlockSpec((block, hid), lambda i: (i, 0)),
        out_shape=jax.ShapeDtypeStruct((e, hid), F32),
    )(ga, gb, gc, vec(b1), w2, vec(b2))


def _edge_mlp(ga, gb, es, wc, b1, w2, b2, block=512):
    e, hid = ga.shape
    grid = (e // block,)
    vec = lambda v: v.reshape(1, -1)
    return pl.pallas_call(
        _edge_body,
        grid=grid,
        in_specs=[
            pl.BlockSpec((block, hid), lambda i: (i, 0)),
            pl.BlockSpec((block, hid), lambda i: (i, 0)),
            pl.BlockSpec((block, hid), lambda i: (i, 0)),
            pl.BlockSpec((hid, hid), lambda i: (0, 0)),
            pl.BlockSpec((1, hid), lambda i: (0, 0)),
            pl.BlockSpec((hid, hid), lambda i: (0, 0)),
            pl.BlockSpec((1, hid), lambda i: (0, 0)),
        ],
        out_specs=pl.BlockSpec((block, hid), lambda i: (i, 0)),
        out_shape=jax.ShapeDtypeStruct((e, hid), F32),
    )(ga, gb, es, wc, vec(b1), w2, vec(b2))


def _node_body(residual, x_ref, s_ref, mx_ref, cnt_ref, batch_ref, u_ref,
               w1_ref, b1_ref, w2_ref, b2_ref, out_ref):
    x = x_ref[...]
    s = s_ref[...]
    cnt = cnt_ref[...][:, :1]
    has = cnt > 0.0
    mx = jnp.where(has, mx_ref[...], 0.0)
    mean = s / jnp.maximum(cnt, 1.0)
    g = u_ref.shape[0]
    oh = (batch_ref[...] == lax.broadcasted_iota(jnp.int32, (1, g), 1)).astype(F32)
    ub = jnp.dot(oh, u_ref[...], preferred_element_type=F32)
    cat = jnp.concatenate([x, s, mx, mean, ub], axis=1)
    h = jnp.maximum(jnp.dot(cat, w1_ref[...], preferred_element_type=F32) + b1_ref[...], 0.0)
    o = jnp.dot(h, w2_ref[...], preferred_element_type=F32) + b2_ref[...]
    if residual:
        o = o + x
    out_ref[...] = o


def _node_mlp(x, s, mx, cnt, batch2d, u, w1, b1, w2, b2, residual, block=400):
    n, hid = x.shape
    g, udim = u.shape
    cin = w1.shape[0]
    grid = (n // block,)
    vec = lambda v: v.reshape(1, -1)
    return pl.pallas_call(
        functools.partial(_node_body, residual),
        grid=grid,
        in_specs=[
            pl.BlockSpec((block, hid), lambda i: (i, 0)),
            pl.BlockSpec((block, hid), lambda i: (i, 0)),
            pl.BlockSpec((block, hid), lambda i: (i, 0)),
            pl.BlockSpec((block, 16), lambda i: (i, 0)),
            pl.BlockSpec((block, 1), lambda i: (i, 0)),
            pl.BlockSpec((g, udim), lambda i: (0, 0)),
            pl.BlockSpec((cin, hid), lambda i: (0, 0)),
            pl.BlockSpec((1, hid), lambda i: (0, 0)),
            pl.BlockSpec((hid, hid), lambda i: (0, 0)),
            pl.BlockSpec((1, hid), lambda i: (0, 0)),
        ],
        out_specs=pl.BlockSpec((block, hid), lambda i: (i, 0)),
        out_shape=jax.ShapeDtypeStruct((n, hid), F32),
    )(x, s, mx, cnt, batch2d, u, w1, vec(b1), w2, vec(b2))


def _pool_body(nblocks, x_ref, batch_ref, u_ref,
               w0_ref, b0_ref, w1_ref, b1_ref, w2_ref, b2_ref, w3_ref, b3_ref,
               out_ref, add_scr, max_scr, cnt_scr):
    i = pl.program_id(0)
    g = u_ref.shape[0]

    @pl.when(i == 0)
    def _init():
        add_scr[...] = jnp.zeros_like(add_scr)
        max_scr[...] = jnp.full_like(max_scr, -jnp.inf)
        cnt_scr[...] = jnp.zeros_like(cnt_scr)

    x = x_ref[...]
    b = batch_ref[...]
    oh = (b == lax.broadcasted_iota(jnp.int32, (1, g), 1)).astype(F32)
    add_scr[...] += jnp.dot(oh.T, x, preferred_element_type=F32)
    cnt_scr[...] += jnp.dot(oh.T, jnp.ones_like(x), preferred_element_type=F32)
    for gg in range(g):
        cand = jnp.max(jnp.where(b == gg, x, -jnp.inf), axis=0, keepdims=True)
        max_scr[pl.ds(gg, 1), :] = jnp.maximum(max_scr[pl.ds(gg, 1), :], cand)

    @pl.when(i == nblocks - 1)
    def _final():
        cnt = cnt_scr[...]
        addp = add_scr[...]
        meanp = addp / jnp.maximum(cnt, 1.0)
        maxp = jnp.where(cnt > 0.0, max_scr[...], 0.0)
        o = jnp.concatenate([addp, meanp, maxp, u_ref[...]], axis=1)
        o = jnp.maximum(jnp.dot(o, w0_ref[...], preferred_element_type=F32) + b0_ref[...], 0.0)
        o = jnp.maximum(jnp.dot(o, w1_ref[...], preferred_element_type=F32) + b1_ref[...], 0.0)
        o = jnp.maximum(jnp.dot(o, w2_ref[...], preferred_element_type=F32) + b2_ref[...], 0.0)
        out_ref[...] = jnp.dot(o, w3_ref[...], preferred_element_type=F32) + b3_ref[...]


def _pool_mlp(x, batch2d, u, out_params, block=400):
    n, hid = x.shape
    g, udim = u.shape
    dim_out = out_params[3]["w"].shape[1]
    nblocks = n // block
    vec = lambda v: v.reshape(1, -1)
    cst = lambda shape: pl.BlockSpec(shape, lambda i: tuple(0 for _ in shape))
    return pl.pallas_call(
        functools.partial(_pool_body, nblocks),
        grid=(nblocks,),
        in_specs=[
            pl.BlockSpec((block, hid), lambda i: (i, 0)),
            pl.BlockSpec((block, 1), lambda i: (i, 0)),
            cst((g, udim)),
            cst((3 * hid + udim, hid)), cst((1, hid)),
            cst((hid, hid)), cst((1, hid)),
            cst((hid, hid)), cst((1, hid)),
            cst((hid, dim_out)), cst((1, dim_out)),
        ],
        out_specs=pl.BlockSpec((g, dim_out), lambda i: (0, 0)),
        out_shape=jax.ShapeDtypeStruct((g, dim_out), F32),
        scratch_shapes=[
            pltpu.VMEM((g, hid), F32),
            pltpu.VMEM((g, hid), F32),
            pltpu.VMEM((g, hid), F32),
        ],
    )(x, batch2d, u,
      out_params[0]["w"], vec(out_params[0]["b"]),
      out_params[1]["w"], vec(out_params[1]["b"]),
      out_params[2]["w"], vec(out_params[2]["b"]),
      out_params[3]["w"], vec(out_params[3]["b"]))


_NC, _NS, _LANES = 2, 16, 16
_NW = _NC * _NS


def _sc_mesh():
    return plsc.VectorSubcoreMesh(core_axis_name="c", subcore_axis_name="s")


def _wid():
    return lax.axis_index("s") * _NC + lax.axis_index("c")


def _gather_rows(tables, index_lists):
    ntab = len(tables)
    e = index_lists[0].shape[0]
    hid = tables[0].shape[1]
    epw = e // _NW
    ch = 80
    nch = epw // ch
    assert epw % ch == 0 and e % _NW == 0

    scratch = (
        [pltpu.VMEM((epw,), jnp.int32) for _ in range(ntab)]
        + [pltpu.VMEM((ch, hid), F32) for _ in range(2 * ntab)]
        + [pltpu.SemaphoreType.DMA for _ in range(2 * ntab)]
        + [pltpu.SemaphoreType.DMA for _ in range(2 * ntab)]
    )

    def body(*refs):
        tabs = refs[:ntab]
        idxs_hbm = refs[ntab:2 * ntab]
        outs = refs[2 * ntab:3 * ntab]
        sc = refs[3 * ntab:]
        idx_v = sc[:ntab]
        bufs = sc[ntab:3 * ntab]
        gsem = sc[3 * ntab:5 * ntab]
        ssem = sc[5 * ntab:7 * ntab]
        w = _wid()
        base = w * epw
        for t in range(ntab):
            pltpu.sync_copy(idxs_hbm[t].at[pl.ds(base, epw)], idx_v[t])

        def fire_gathers(k, b):
            for t in range(ntab):
                pltpu.async_copy(
                    tabs[t].at[idx_v[t].at[pl.ds(k * ch, ch)]],
                    bufs[2 * t + b], gsem[2 * t + b])

        def wait_gathers(b):
            for t in range(ntab):
                pltpu.make_async_copy(
                    tabs[t].at[idx_v[t].at[pl.ds(0, ch)]],
                    bufs[2 * t + b], gsem[2 * t + b]).wait()

        def fire_stores(k, b):
            for t in range(ntab):
                pltpu.async_copy(
                    bufs[2 * t + b], outs[t].at[pl.ds(base + k * ch, ch)],
                    ssem[2 * t + b])

        def wait_stores(b):
            for t in range(ntab):
                pltpu.make_async_copy(
                    bufs[2 * t + b], outs[t].at[pl.ds(base, ch)],
                    ssem[2 * t + b]).wait()

        fire_gathers(0, 0)

        def step(k2, carry):
            for b in (0, 1):
                k = k2 * 2 + b

                @pl.when(k < nch)
                def _():
                    wait_gathers(b)

                    @pl.when(k + 1 < nch)
                    def _():
                        @pl.when(k >= 1)
                        def _():
                            wait_stores(1 - b)
                        fire_gathers(k + 1, 1 - b)

                    fire_stores(k, b)
            return carry

        lax.fori_loop(0, (nch + 1) // 2, step, 0)
        wait_stores((nch - 1) % 2)
        wait_stores((nch - 2) % 2)

    out_type = [jax.ShapeDtypeStruct((e, hid), F32) for _ in range(ntab)]
    fn = pl.kernel(body, out_type=out_type, mesh=_sc_mesh(),
                   scratch_types=scratch)
    res = fn(*tables, *index_lists)
    return list(res) if ntab > 1 else [res]


def _segment_reduce(new_e, col_s, starts_pairs, n_pad):
    e, hid = new_e.shape
    rpt = n_pad // _NW
    ce = 96
    nf = hid // _LANES

    scratch = (
        [pltpu.VMEM((rpt, hid), F32),
         pltpu.VMEM((rpt, hid), F32),
         pltpu.VMEM((rpt * _LANES,), F32),
         pltpu.VMEM((_LANES,), jnp.int32)]
        + [pltpu.VMEM((ce, hid), F32) for _ in range(2)]
        + [pltpu.VMEM((ce,), jnp.int32) for _ in range(2)]
        + [pltpu.SemaphoreType.DMA for _ in range(2)]
    )

    def body(vals_hbm, col_hbm, sp_hbm, s_out, mx_out, cnt_out,
             sumtbl, maxtbl, cnttbl, sbuf, vb0, vb1, cb0, cb1, sem0, sem1):
        vb = (vb0, vb1)
        cb = (cb0, cb1)
        sem = (sem0, sem1)
        w = _wid()
        node_base = w * rpt
        pltpu.sync_copy(sp_hbm.at[pl.ds(w * _LANES, _LANES)], sbuf)
        sv = sbuf[pl.ds(0, _LANES)]
        start = sv[0]
        end = sv[1]
        start8 = (start // 8) * 8
        nch = lax.div(end - start8 + (ce - 1), ce)

        zeros = jnp.zeros((_LANES,), F32)
        ninf = jnp.full((_LANES,), -jnp.inf, F32)

        def init_row(r, carry):
            for f in range(nf):
                sl = pl.ds(f * _LANES, _LANES)
                sumtbl[r, sl] = zeros
                maxtbl[r, sl] = ninf
            cnttbl[pl.ds(r * _LANES, _LANES)] = zeros
            return carry

        lax.fori_loop(0, rpt, init_row, 0)

        emax = e - ce

        def fire(k, b):
            eb = jnp.minimum(start8 + k * ce, emax)
            pltpu.async_copy(col_hbm.at[pl.ds(eb, ce)], cb[b], sem[b])
            pltpu.async_copy(vals_hbm.at[pl.ds(eb, ce)], vb[b], sem[b])

        def wait(b):
            pltpu.make_async_copy(col_hbm.at[pl.ds(0, ce)], cb[b], sem[b]).wait()
            pltpu.make_async_copy(vals_hbm.at[pl.ds(0, ce)], vb[b], sem[b]).wait()

        @pl.when(nch > 0)
        def _():
            fire(0, 0)

        def process(k, b):
            g0 = start8 + k * ce
            eb = jnp.minimum(g0, emax)
            lo = jnp.maximum(start, g0) - eb
            hi = jnp.minimum(end, g0 + ce) - eb
            one = jnp.full((_LANES,), 1.0, F32)

            def group(gi, carry):
                base = gi * _LANES
                cvec = cb[b][pl.ds(base, _LANES)] - node_base
                for j in range(_LANES):
                    ei = base + j

                    @pl.when((ei >= lo) & (ei < hi))
                    def _():
                        c = cvec[j]
                        for f in range(nf):
                            sl = pl.ds(f * _LANES, _LANES)
                            v = vb[b][ei, sl]
                            maxtbl[c, sl] = jnp.maximum(maxtbl[c, sl], v)
                            sumtbl[c, sl] = sumtbl[c, sl] + v
                        csl = pl.ds(c * _LANES, _LANES)
                        cnttbl[csl] = cnttbl[csl] + one

                return carry

            lax.fori_loop(0, ce // _LANES, group, 0)

        def step(k2, carry):
            for b in (0, 1):
                k = k2 * 2 + b

                @pl.when(k < nch)
                def _():
                    wait(b)

                    @pl.when(k + 1 < nch)
                    def _():
                        fire(k + 1, 1 - b)

                    process(k, b)
            return carry

        lax.fori_loop(0, (nch + 1) // 2, step, 0)

        pltpu.sync_copy(sumtbl, s_out.at[pl.ds(node_base, rpt)])
        pltpu.sync_copy(maxtbl, mx_out.at[pl.ds(node_base, rpt)])
        pltpu.sync_copy(cnttbl, cnt_out.at[pl.ds(node_base * _LANES, rpt * _LANES)])

    out_type = [
        jax.ShapeDtypeStruct((n_pad, hid), F32),
        jax.ShapeDtypeStruct((n_pad, hid), F32),
        jax.ShapeDtypeStruct((n_pad * _LANES,), F32),
    ]
    fn = pl.kernel(body, out_type=out_type, mesh=_sc_mesh(),
                   scratch_types=scratch)
    return fn(new_e, col_s, starts_pairs)


def kernel(x, edge_index, edge_attr, u, batch, params):
    n, f_node = x.shape
    e = edge_index.shape[1]
    hid = params["layers"][0]["e2"]["w"].shape[1]
    row, col = edge_index[0], edge_index[1]

    n_tiles = 32
    rows_per_tile = ((math.ceil(n / n_tiles) + 7) // 8) * 8
    n_pad = rows_per_tile * n_tiles
    perm = jnp.argsort(col).astype(jnp.int32)
    row_s = row[perm]
    col_s = col[perm]
    bounds = jnp.arange(n_tiles + 1, dtype=jnp.int32) * rows_per_tile
    starts = jnp.searchsorted(col_s, bounds, side="left").astype(jnp.int32)
    starts_pairs = jnp.pad(
        jnp.stack([starts[:-1], starts[1:]], axis=1), ((0, 0), (0, 14))).reshape(-1)

    batch2d = batch.reshape(n, 1)

    e_s = None
    for i, lp in enumerate(params["layers"]):
        residual = i > 0
        w1 = lp["e1"]["w"]
        wa, wb, wc = w1[:f_node], w1[f_node:2 * f_node], w1[2 * f_node:]
        xa, xb = _node_projections(x, wa, wb)
        if i == 0:
            ec0 = _matmul(edge_attr, wc)
            ga, gb, gc = _gather_rows([xa, xb, ec0], [row_s, col_s, perm])
            new_e = _edge_mlp_l0(ga, gb, gc, lp["e1"]["b"], lp["e2"]["w"], lp["e2"]["b"])
        else:
            ga, gb = _gather_rows([xa, xb], [row_s, col_s])
            new_e = _edge_mlp(ga, gb, e_s, wc, lp["e1"]["b"], lp["e2"]["w"], lp["e2"]["b"])
        e_s = new_e
        s, mx, cnt_flat = _segment_reduce(new_e, col_s, starts_pairs, n_pad)
        s, mx, cnt16 = s[:n], mx[:n], cnt_flat.reshape(n_pad, 16)[:n]
        x = _node_mlp(x, s, mx, cnt16, batch2d, u,
                      lp["n1"]["w"], lp["n1"]["b"], lp["n2"]["w"], lp["n2"]["b"],
                      residual)
    return _pool_mlp(x, batch2d, u, params["out"])


# speedup vs baseline: 1.8616x; 1.1861x over previous
import functools
import math

import jax
import jax.numpy as jnp
from jax import lax
from jax.experimental import pallas as pl
from jax.experimental.pallas import tpu as pltpu
from jax.experimental.pallas import tpu_sc as plsc

F32 = jnp.float32


def _prep_body(x_ref, wab_ref, xa_ref, xb_ref):
    y = jnp.dot(x_ref[...], wab_ref[...], preferred_element_type=F32)
    h = y.shape[1] // 2
    xa_ref[...] = y[:, :h]
    xb_ref[...] = y[:, h:]


def _node_projections(x, wa, wb, block=400):
    n, f = x.shape
    hid = wa.shape[1]
    wab = jnp.concatenate([wa, wb], axis=1)
    grid = (n // block,)
    return pl.pallas_call(
        _prep_body,
        grid=grid,
        in_specs=[
            pl.BlockSpec((block, f), lambda i: (i, 0)),
            pl.BlockSpec((f, 2 * hid), lambda i: (0, 0)),
        ],
        out_specs=[
            pl.BlockSpec((block, hid), lambda i: (i, 0)),
            pl.BlockSpec((block, hid), lambda i: (i, 0)),
        ],
        out_shape=[
            jax.ShapeDtypeStruct((n, hid), F32),
            jax.ShapeDtypeStruct((n, hid), F32),
        ],
    )(x, wab)


def _matmul_body(x_ref, w_ref, out_ref):
    out_ref[...] = jnp.dot(x_ref[...], w_ref[...], preferred_element_type=F32)


def _matmul(x, w, block=512):
    r, k = x.shape
    f = w.shape[1]
    return pl.pallas_call(
        _matmul_body,
        grid=(r // block,),
        in_specs=[
            pl.BlockSpec((block, k), lambda i: (i, 0)),
            pl.BlockSpec((k, f), lambda i: (0, 0)),
        ],
        out_specs=pl.BlockSpec((block, f), lambda i: (i, 0)),
        out_shape=jax.ShapeDtypeStruct((r, f), F32),
    )(x, w)


def _edge0_body(ga_ref, gb_ref, gc_ref, b1_ref, w2_ref, b2_ref, out_ref):
    h = jnp.maximum(ga_ref[...] + gb_ref[...] + gc_ref[...] + b1_ref[...], 0.0)
    out_ref[...] = jnp.dot(h, w2_ref[...], preferred_element_type=F32) + b2_ref[...]


def _edge_body(ga_ref, gb_ref, es_ref, wc_ref, b1_ref, w2_ref, b2_ref, out_ref):
    es = es_ref[...]
    ec = jnp.dot(es, wc_ref[...], preferred_element_type=F32)
    h = jnp.maximum(ga_ref[...] + gb_ref[...] + ec + b1_ref[...], 0.0)
    out_ref[...] = jnp.dot(h, w2_ref[...], preferred_element_type=F32) + b2_ref[...] + es


def _edge_mlp_l0(ga, gb, gc, b1, w2, b2, block=512):
    e, hid = ga.shape
    grid = (e // block,)
    vec = lambda v: v.reshape(1, -1)
    return pl.pallas_call(
        _edge0_body,
        grid=grid,
        in_specs=[
            pl.BlockSpec((block, hid), lambda i: (i, 0)),
            pl.BlockSpec((block, hid), lambda i: (i, 0)),
            pl.BlockSpec((block, hid), lambda i: (i, 0)),
            pl.BlockSpec((1, hid), lambda i: (0, 0)),
            pl.BlockSpec((hid, hid), lambda i: (0, 0)),
            pl.BlockSpec((1, hid), lambda i: (0, 0)),
        ],
        out_specs=pl.BlockSpec((block, hid), lambda i: (i, 0)),
        out_shape=jax.ShapeDtypeStruct((e, hid), F32),
    )(ga, gb, gc, vec(b1), w2, vec(b2))


def _edge_mlp(ga, gb, es, wc, b1, w2, b2, block=512):
    e, hid = ga.shape
    grid = (e // block,)
    vec = lambda v: v.reshape(1, -1)
    return pl.pallas_call(
        _edge_body,
        grid=grid,
        in_specs=[
            pl.BlockSpec((block, hid), lambda i: (i, 0)),
            pl.BlockSpec((block, hid), lambda i: (i, 0)),
            pl.BlockSpec((block, hid), lambda i: (i, 0)),
            pl.BlockSpec((hid, hid), lambda i: (0, 0)),
            pl.BlockSpec((1, hid), lambda i: (0, 0)),
            pl.BlockSpec((hid, hid), lambda i: (0, 0)),
            pl.BlockSpec((1, hid), lambda i: (0, 0)),
        ],
        out_specs=pl.BlockSpec((block, hid), lambda i: (i, 0)),
        out_shape=jax.ShapeDtypeStruct((e, hid), F32),
    )(ga, gb, es, wc, vec(b1), w2, vec(b2))


def _node_body(residual, x_ref, s_ref, mx_ref, cnt_ref, batch_ref, u_ref,
               w1_ref, b1_ref, w2_ref, b2_ref, out_ref):
    x = x_ref[...]
    s = s_ref[...]
    cnt = cnt_ref[...][:, :1]
    has = cnt > 0.0
    mx = jnp.where(has, mx_ref[...], 0.0)
    mean = s / jnp.maximum(cnt, 1.0)
    g = u_ref.shape[0]
    oh = (batch_ref[...] == lax.broadcasted_iota(jnp.int32, (1, g), 1)).astype(F32)
    ub = jnp.dot(oh, u_ref[...], preferred_element_type=F32)
    cat = jnp.concatenate([x, s, mx, mean, ub], axis=1)
    h = jnp.maximum(jnp.dot(cat, w1_ref[...], preferred_element_type=F32) + b1_ref[...], 0.0)
    o = jnp.dot(h, w2_ref[...], preferred_element_type=F32) + b2_ref[...]
    if residual:
        o = o + x
    out_ref[...] = o


def _node_mlp(x, s, mx, cnt, batch2d, u, w1, b1, w2, b2, residual, block=400):
    n, hid = x.shape
    g, udim = u.shape
    cin = w1.shape[0]
    grid = (n // block,)
    vec = lambda v: v.reshape(1, -1)
    return pl.pallas_call(
        functools.partial(_node_body, residual),
        grid=grid,
        in_specs=[
            pl.BlockSpec((block, hid), lambda i: (i, 0)),
            pl.BlockSpec((block, hid), lambda i: (i, 0)),
            pl.BlockSpec((block, hid), lambda i: (i, 0)),
            pl.BlockSpec((block, 16), lambda i: (i, 0)),
            pl.BlockSpec((block, 1), lambda i: (i, 0)),
            pl.BlockSpec((g, udim), lambda i: (0, 0)),
            pl.BlockSpec((cin, hid), lambda i: (0, 0)),
            pl.BlockSpec((1, hid), lambda i: (0, 0)),
            pl.BlockSpec((hid, hid), lambda i: (0, 0)),
            pl.BlockSpec((1, hid), lambda i: (0, 0)),
        ],
        out_specs=pl.BlockSpec((block, hid), lambda i: (i, 0)),
        out_shape=jax.ShapeDtypeStruct((n, hid), F32),
    )(x, s, mx, cnt, batch2d, u, w1, vec(b1), w2, vec(b2))


def _pool_body(nblocks, x_ref, batch_ref, u_ref,
               w0_ref, b0_ref, w1_ref, b1_ref, w2_ref, b2_ref, w3_ref, b3_ref,
               out_ref, add_scr, max_scr, cnt_scr):
    i = pl.program_id(0)
    g = u_ref.shape[0]

    @pl.when(i == 0)
    def _init():
        add_scr[...] = jnp.zeros_like(add_scr)
        max_scr[...] = jnp.full_like(max_scr, -jnp.inf)
        cnt_scr[...] = jnp.zeros_like(cnt_scr)

    x = x_ref[...]
    b = batch_ref[...]
    oh = (b == lax.broadcasted_iota(jnp.int32, (1, g), 1)).astype(F32)
    add_scr[...] += jnp.dot(oh.T, x, preferred_element_type=F32)
    cnt_scr[...] += jnp.dot(oh.T, jnp.ones_like(x), preferred_element_type=F32)
    for gg in range(g):
        cand = jnp.max(jnp.where(b == gg, x, -jnp.inf), axis=0, keepdims=True)
        max_scr[pl.ds(gg, 1), :] = jnp.maximum(max_scr[pl.ds(gg, 1), :], cand)

    @pl.when(i == nblocks - 1)
    def _final():
        cnt = cnt_scr[...]
        addp = add_scr[...]
        meanp = addp / jnp.maximum(cnt, 1.0)
        maxp = jnp.where(cnt > 0.0, max_scr[...], 0.0)
        o = jnp.concatenate([addp, meanp, maxp, u_ref[...]], axis=1)
        o = jnp.maximum(jnp.dot(o, w0_ref[...], preferred_element_type=F32) + b0_ref[...], 0.0)
        o = jnp.maximum(jnp.dot(o, w1_ref[...], preferred_element_type=F32) + b1_ref[...], 0.0)
        o = jnp.maximum(jnp.dot(o, w2_ref[...], preferred_element_type=F32) + b2_ref[...], 0.0)
        out_ref[...] = jnp.dot(o, w3_ref[...], preferred_element_type=F32) + b3_ref[...]


def _pool_mlp(x, batch2d, u, out_params, block=400):
    n, hid = x.shape
    g, udim = u.shape
    dim_out = out_params[3]["w"].shape[1]
    nblocks = n // block
    vec = lambda v: v.reshape(1, -1)
    cst = lambda shape: pl.BlockSpec(shape, lambda i: tuple(0 for _ in shape))
    return pl.pallas_call(
        functools.partial(_pool_body, nblocks),
        grid=(nblocks,),
        in_specs=[
            pl.BlockSpec((block, hid), lambda i: (i, 0)),
            pl.BlockSpec((block, 1), lambda i: (i, 0)),
            cst((g, udim)),
            cst((3 * hid + udim, hid)), cst((1, hid)),
            cst((hid, hid)), cst((1, hid)),
            cst((hid, hid)), cst((1, hid)),
            cst((hid, dim_out)), cst((1, dim_out)),
        ],
        out_specs=pl.BlockSpec((g, dim_out), lambda i: (0, 0)),
        out_shape=jax.ShapeDtypeStruct((g, dim_out), F32),
        scratch_shapes=[
            pltpu.VMEM((g, hid), F32),
            pltpu.VMEM((g, hid), F32),
            pltpu.VMEM((g, hid), F32),
        ],
    )(x, batch2d, u,
      out_params[0]["w"], vec(out_params[0]["b"]),
      out_params[1]["w"], vec(out_params[1]["b"]),
      out_params[2]["w"], vec(out_params[2]["b"]),
      out_params[3]["w"], vec(out_params[3]["b"]))


_NC, _NS, _LANES = 2, 16, 16
_NW = _NC * _NS


def _sc_mesh():
    return plsc.VectorSubcoreMesh(core_axis_name="c", subcore_axis_name="s")


def _wid():
    return lax.axis_index("s") * _NC + lax.axis_index("c")


def _gather_rows(tables, index_lists):
    ntab = len(tables)
    e = index_lists[0].shape[0]
    hid = tables[0].shape[1]
    epw = e // _NW
    ch = 80
    nch = epw // ch
    assert epw % ch == 0 and e % _NW == 0

    scratch = (
        [pltpu.VMEM((epw,), jnp.int32) for _ in range(ntab)]
        + [pltpu.VMEM((ch, hid), F32) for _ in range(2 * ntab)]
        + [pltpu.SemaphoreType.DMA for _ in range(2 * ntab)]
        + [pltpu.SemaphoreType.DMA for _ in range(2 * ntab)]
    )

    def body(*refs):
        tabs = refs[:ntab]
        idxs_hbm = refs[ntab:2 * ntab]
        outs = refs[2 * ntab:3 * ntab]
        sc = refs[3 * ntab:]
        idx_v = sc[:ntab]
        bufs = sc[ntab:3 * ntab]
        gsem = sc[3 * ntab:5 * ntab]
        ssem = sc[5 * ntab:7 * ntab]
        w = _wid()
        base = w * epw
        for t in range(ntab):
            pltpu.sync_copy(idxs_hbm[t].at[pl.ds(base, epw)], idx_v[t])

        def fire_gathers(k, b):
            for t in range(ntab):
                pltpu.async_copy(
                    tabs[t].at[idx_v[t].at[pl.ds(k * ch, ch)]],
                    bufs[2 * t + b], gsem[2 * t + b])

        def wait_gathers(b):
            for t in range(ntab):
                pltpu.make_async_copy(
                    tabs[t].at[idx_v[t].at[pl.ds(0, ch)]],
                    bufs[2 * t + b], gsem[2 * t + b]).wait()

        def fire_stores(k, b):
            for t in range(ntab):
                pltpu.async_copy(
                    bufs[2 * t + b], outs[t].at[pl.ds(base + k * ch, ch)],
                    ssem[2 * t + b])

        def wait_stores(b):
            for t in range(ntab):
                pltpu.make_async_copy(
                    bufs[2 * t + b], outs[t].at[pl.ds(base, ch)],
                    ssem[2 * t + b]).wait()

        fire_gathers(0, 0)

        def step(k2, carry):
            for b in (0, 1):
                k = k2 * 2 + b

                @pl.when(k < nch)
                def _():
                    wait_gathers(b)

                    @pl.when(k + 1 < nch)
                    def _():
                        @pl.when(k >= 1)
                        def _():
                            wait_stores(1 - b)
                        fire_gathers(k + 1, 1 - b)

                    fire_stores(k, b)
            return carry

        lax.fori_loop(0, (nch + 1) // 2, step, 0)
        wait_stores((nch - 1) % 2)
        wait_stores((nch - 2) % 2)

    out_type = [jax.ShapeDtypeStruct((e, hid), F32) for _ in range(ntab)]
    fn = pl.kernel(body, out_type=out_type, mesh=_sc_mesh(),
                   scratch_types=scratch)
    res = fn(*tables, *index_lists)
    return list(res) if ntab > 1 else [res]


def _segment_reduce(new_e, col_s, starts_pairs, n_pad):
    e, hid = new_e.shape
    rpt = n_pad // _NW
    ce = 96
    nf = hid // _LANES

    scratch = (
        [pltpu.VMEM((rpt + 1, hid), F32),
         pltpu.VMEM((rpt + 1, hid), F32),
         pltpu.VMEM(((rpt + 1) * _LANES,), F32),
         pltpu.VMEM((_LANES,), jnp.int32)]
        + [pltpu.VMEM((ce, hid), F32) for _ in range(2)]
        + [pltpu.VMEM((ce,), jnp.int32) for _ in range(2)]
        + [pltpu.SemaphoreType.DMA for _ in range(2)]
    )

    def body(vals_hbm, col_hbm, sp_hbm, s_out, mx_out, cnt_out,
             sumtbl, maxtbl, cnttbl, sbuf, vb0, vb1, cb0, cb1, sem0, sem1):
        vb = (vb0, vb1)
        cb = (cb0, cb1)
        sem = (sem0, sem1)
        w = _wid()
        node_base = w * rpt
        pltpu.sync_copy(sp_hbm.at[pl.ds(w * _LANES, _LANES)], sbuf)
        sv = sbuf[pl.ds(0, _LANES)]
        start = sv[0]
        end = sv[1]
        start8 = (start // 8) * 8
        nch = lax.div(end - start8 + (ce - 1), ce)

        zeros = jnp.zeros((_LANES,), F32)
        ninf = jnp.full((_LANES,), -jnp.inf, F32)

        def init_row(r, carry):
            for f in range(nf):
                sl = pl.ds(f * _LANES, _LANES)
                sumtbl[r, sl] = zeros
                maxtbl[r, sl] = ninf
            cnttbl[pl.ds(r * _LANES, _LANES)] = zeros
            return carry

        lax.fori_loop(0, rpt + 1, init_row, 0)

        emax = e - ce

        def fire(k, b):
            eb = jnp.minimum(start8 + k * ce, emax)
            pltpu.async_copy(col_hbm.at[pl.ds(eb, ce)], cb[b], sem[b])
            pltpu.async_copy(vals_hbm.at[pl.ds(eb, ce)], vb[b], sem[b])

        def wait(b):
            pltpu.make_async_copy(col_hbm.at[pl.ds(0, ce)], cb[b], sem[b]).wait()
            pltpu.make_async_copy(vals_hbm.at[pl.ds(0, ce)], vb[b], sem[b]).wait()

        @pl.when(nch > 0)
        def _():
            fire(0, 0)

        def flush(cur, cnt, accs, accm):
            @pl.when(cur >= 0)
            def _():
                for f in range(nf):
                    sl = pl.ds(f * _LANES, _LANES)
                    sumtbl[cur, sl] = sumtbl[cur, sl] + accs[f]
                    maxtbl[cur, sl] = jnp.maximum(maxtbl[cur, sl], accm[f])
                csl = pl.ds(cur * _LANES, _LANES)
                cnttbl[csl] = cnttbl[csl] + cnt

        def process(k, b):
            g0 = start8 + k * ce
            eb = jnp.minimum(g0, emax)
            lo = jnp.maximum(start, g0) - eb
            hi = jnp.minimum(end, g0 + ce) - eb
            zero_v = jnp.zeros((_LANES,), F32)
            ninf_v = jnp.full((_LANES,), -jnp.inf, F32)
            carry0 = ((jnp.int32(-1), jnp.float32(0.0))
                      + tuple(zero_v for _ in range(nf))
                      + tuple(ninf_v for _ in range(nf)))

            def group(gi, carry):
                base = gi * _LANES
                cvec = cb[b][pl.ds(base, _LANES)] - node_base
                for j in range(_LANES):
                    ei = base + j
                    active = (ei >= lo) & (ei < hi)
                    c = jnp.where(active, cvec[j], rpt)
                    vlist = [vb[b][ei, pl.ds(f * _LANES, _LANES)]
                             for f in range(nf)]
                    cur, cnt = carry[0], carry[1]
                    accs = carry[2:2 + nf]
                    accm = carry[2 + nf:]
                    is_new = c != cur

                    @pl.when(is_new)
                    def _():
                        flush(cur, cnt, accs, accm)

                    carry = ((c, jnp.where(is_new, 1.0, cnt + 1.0))
                             + tuple(jnp.where(is_new, v, a + v)
                                     for a, v in zip(accs, vlist))
                             + tuple(jnp.where(is_new, v, jnp.maximum(m, v))
                                     for m, v in zip(accm, vlist)))
                return carry

            fin = lax.fori_loop(0, ce // _LANES, group, carry0)
            flush(fin[0], fin[1], fin[2:2 + nf], fin[2 + nf:])

        def step(k2, carry):
            for b in (0, 1):
                k = k2 * 2 + b

                @pl.when(k < nch)
                def _():
                    wait(b)

                    @pl.when(k + 1 < nch)
                    def _():
                        fire(k + 1, 1 - b)

                    process(k, b)
            return carry

        lax.fori_loop(0, (nch + 1) // 2, step, 0)

        pltpu.sync_copy(sumtbl.at[pl.ds(0, rpt)], s_out.at[pl.ds(node_base, rpt)])
        pltpu.sync_copy(maxtbl.at[pl.ds(0, rpt)], mx_out.at[pl.ds(node_base, rpt)])
        pltpu.sync_copy(cnttbl.at[pl.ds(0, rpt * _LANES)],
                        cnt_out.at[pl.ds(node_base * _LANES, rpt * _LANES)])

    out_type = [
        jax.ShapeDtypeStruct((n_pad, hid), F32),
        jax.ShapeDtypeStruct((n_pad, hid), F32),
        jax.ShapeDtypeStruct((n_pad * _LANES,), F32),
    ]
    fn = pl.kernel(body, out_type=out_type, mesh=_sc_mesh(),
                   scratch_types=scratch)
    return fn(new_e, col_s, starts_pairs)


def kernel(x, edge_index, edge_attr, u, batch, params):
    n, f_node = x.shape
    e = edge_index.shape[1]
    hid = params["layers"][0]["e2"]["w"].shape[1]
    row, col = edge_index[0], edge_index[1]

    n_tiles = 32
    rows_per_tile = ((math.ceil(n / n_tiles) + 7) // 8) * 8
    n_pad = rows_per_tile * n_tiles
    perm = jnp.argsort(col).astype(jnp.int32)
    row_s = row[perm]
    col_s = col[perm]
    bounds = jnp.arange(n_tiles + 1, dtype=jnp.int32) * rows_per_tile
    starts = jnp.searchsorted(col_s, bounds, side="left").astype(jnp.int32)
    starts_pairs = jnp.pad(
        jnp.stack([starts[:-1], starts[1:]], axis=1), ((0, 0), (0, 14))).reshape(-1)

    batch2d = batch.reshape(n, 1)

    e_s = None
    for i, lp in enumerate(params["layers"]):
        residual = i > 0
        w1 = lp["e1"]["w"]
        wa, wb, wc = w1[:f_node], w1[f_node:2 * f_node], w1[2 * f_node:]
        xa, xb = _node_projections(x, wa, wb)
        if i == 0:
            ec0 = _matmul(edge_attr, wc)
            ga, gb, gc = _gather_rows([xa, xb, ec0], [row_s, col_s, perm])
            new_e = _edge_mlp_l0(ga, gb, gc, lp["e1"]["b"], lp["e2"]["w"], lp["e2"]["b"])
        else:
            ga, gb = _gather_rows([xa, xb], [row_s, col_s])
            new_e = _edge_mlp(ga, gb, e_s, wc, lp["e1"]["b"], lp["e2"]["w"], lp["e2"]["b"])
        e_s = new_e
        s, mx, cnt_flat = _segment_reduce(new_e, col_s, starts_pairs, n_pad)
        s, mx, cnt16 = s[:n], mx[:n], cnt_flat.reshape(n_pad, 16)[:n]
        x = _node_mlp(x, s, mx, cnt16, batch2d, u,
                      lp["n1"]["w"], lp["n1"]["b"], lp["n2"]["w"], lp["n2"]["b"],
                      residual)
    return _pool_mlp(x, batch2d, u, params["out"])


# speedup vs baseline: 1.9300x; 1.0368x over previous
import functools
import math

import jax
import jax.numpy as jnp
from jax import lax
from jax.experimental import pallas as pl
from jax.experimental.pallas import tpu as pltpu
from jax.experimental.pallas import tpu_sc as plsc

F32 = jnp.float32


def _prep_body(x_ref, wab_ref, xa_ref, xb_ref):
    y = jnp.dot(x_ref[...], wab_ref[...], preferred_element_type=F32)
    h = y.shape[1] // 2
    xa_ref[...] = y[:, :h]
    xb_ref[...] = y[:, h:]


def _node_projections(x, wa, wb, block=400):
    n, f = x.shape
    hid = wa.shape[1]
    wab = jnp.concatenate([wa, wb], axis=1)
    grid = (n // block,)
    return pl.pallas_call(
        _prep_body,
        grid=grid,
        in_specs=[
            pl.BlockSpec((block, f), lambda i: (i, 0)),
            pl.BlockSpec((f, 2 * hid), lambda i: (0, 0)),
        ],
        out_specs=[
            pl.BlockSpec((block, hid), lambda i: (i, 0)),
            pl.BlockSpec((block, hid), lambda i: (i, 0)),
        ],
        out_shape=[
            jax.ShapeDtypeStruct((n, hid), F32),
            jax.ShapeDtypeStruct((n, hid), F32),
        ],
    )(x, wab)


def _matmul_body(x_ref, w_ref, out_ref):
    out_ref[...] = jnp.dot(x_ref[...], w_ref[...], preferred_element_type=F32)


def _matmul(x, w, block=512):
    r, k = x.shape
    f = w.shape[1]
    return pl.pallas_call(
        _matmul_body,
        grid=(r // block,),
        in_specs=[
            pl.BlockSpec((block, k), lambda i: (i, 0)),
            pl.BlockSpec((k, f), lambda i: (0, 0)),
        ],
        out_specs=pl.BlockSpec((block, f), lambda i: (i, 0)),
        out_shape=jax.ShapeDtypeStruct((r, f), F32),
    )(x, w)


def _edge0_body(ga_ref, gb_ref, gc_ref, b1_ref, w2_ref, b2_ref, out_ref):
    h = jnp.maximum(ga_ref[...] + gb_ref[...] + gc_ref[...] + b1_ref[...], 0.0)
    out_ref[...] = jnp.dot(h, w2_ref[...], preferred_element_type=F32) + b2_ref[...]


def _edge_body(ga_ref, gb_ref, es_ref, wc_ref, b1_ref, w2_ref, b2_ref, out_ref):
    es = es_ref[...]
    ec = jnp.dot(es, wc_ref[...], preferred_element_type=F32)
    h = jnp.maximum(ga_ref[...] + gb_ref[...] + ec + b1_ref[...], 0.0)
    out_ref[...] = jnp.dot(h, w2_ref[...], preferred_element_type=F32) + b2_ref[...] + es


def _edge_mlp_l0(ga, gb, gc, b1, w2, b2, block=512):
    e, hid = ga.shape
    grid = (e // block,)
    vec = lambda v: v.reshape(1, -1)
    return pl.pallas_call(
        _edge0_body,
        grid=grid,
        in_specs=[
            pl.BlockSpec((block, hid), lambda i: (i, 0)),
            pl.BlockSpec((block, hid), lambda i: (i, 0)),
            pl.BlockSpec((block, hid), lambda i: (i, 0)),
            pl.BlockSpec((1, hid), lambda i: (0, 0)),
            pl.BlockSpec((hid, hid), lambda i: (0, 0)),
            pl.BlockSpec((1, hid), lambda i: (0, 0)),
        ],
        out_specs=pl.BlockSpec((block, hid), lambda i: (i, 0)),
        out_shape=jax.ShapeDtypeStruct((e, hid), F32),
    )(ga, gb, gc, vec(b1), w2, vec(b2))


def _edge_mlp(ga, gb, es, wc, b1, w2, b2, block=512):
    e, hid = ga.shape
    grid = (e // block,)
    vec = lambda v: v.reshape(1, -1)
    return pl.pallas_call(
        _edge_body,
        grid=grid,
        in_specs=[
            pl.BlockSpec((block, hid), lambda i: (i, 0)),
            pl.BlockSpec((block, hid), lambda i: (i, 0)),
            pl.BlockSpec((block, hid), lambda i: (i, 0)),
            pl.BlockSpec((hid, hid), lambda i: (0, 0)),
            pl.BlockSpec((1, hid), lambda i: (0, 0)),
            pl.BlockSpec((hid, hid), lambda i: (0, 0)),
            pl.BlockSpec((1, hid), lambda i: (0, 0)),
        ],
        out_specs=pl.BlockSpec((block, hid), lambda i: (i, 0)),
        out_shape=jax.ShapeDtypeStruct((e, hid), F32),
    )(ga, gb, es, wc, vec(b1), w2, vec(b2))


def _node_body(residual, x_ref, s_ref, mx_ref, cnt_ref, batch_ref, u_ref,
               w1_ref, b1_ref, w2_ref, b2_ref, out_ref):
    x = x_ref[...]
    s = s_ref[...]
    cnt = cnt_ref[...][:, :1]
    has = cnt > 0.0
    mx = jnp.where(has, mx_ref[...], 0.0)
    mean = s / jnp.maximum(cnt, 1.0)
    g = u_ref.shape[0]
    oh = (batch_ref[...] == lax.broadcasted_iota(jnp.int32, (1, g), 1)).astype(F32)
    ub = jnp.dot(oh, u_ref[...], preferred_element_type=F32)
    cat = jnp.concatenate([x, s, mx, mean, ub], axis=1)
    h = jnp.maximum(jnp.dot(cat, w1_ref[...], preferred_element_type=F32) + b1_ref[...], 0.0)
    o = jnp.dot(h, w2_ref[...], preferred_element_type=F32) + b2_ref[...]
    if residual:
        o = o + x
    out_ref[...] = o


def _node_mlp(x, s, mx, cnt, batch2d, u, w1, b1, w2, b2, residual, block=400):
    n, hid = x.shape
    g, udim = u.shape
    cin = w1.shape[0]
    grid = (n // block,)
    vec = lambda v: v.reshape(1, -1)
    return pl.pallas_call(
        functools.partial(_node_body, residual),
        grid=grid,
        in_specs=[
            pl.BlockSpec((block, hid), lambda i: (i, 0)),
            pl.BlockSpec((block, hid), lambda i: (i, 0)),
            pl.BlockSpec((block, hid), lambda i: (i, 0)),
            pl.BlockSpec((block, 16), lambda i: (i, 0)),
            pl.BlockSpec((block, 1), lambda i: (i, 0)),
            pl.BlockSpec((g, udim), lambda i: (0, 0)),
            pl.BlockSpec((cin, hid), lambda i: (0, 0)),
            pl.BlockSpec((1, hid), lambda i: (0, 0)),
            pl.BlockSpec((hid, hid), lambda i: (0, 0)),
            pl.BlockSpec((1, hid), lambda i: (0, 0)),
        ],
        out_specs=pl.BlockSpec((block, hid), lambda i: (i, 0)),
        out_shape=jax.ShapeDtypeStruct((n, hid), F32),
    )(x, s, mx, cnt, batch2d, u, w1, vec(b1), w2, vec(b2))


def _pool_body(nblocks, x_ref, batch_ref, u_ref,
               w0_ref, b0_ref, w1_ref, b1_ref, w2_ref, b2_ref, w3_ref, b3_ref,
               out_ref, add_scr, max_scr, cnt_scr):
    i = pl.program_id(0)
    g = u_ref.shape[0]

    @pl.when(i == 0)
    def _init():
        add_scr[...] = jnp.zeros_like(add_scr)
        max_scr[...] = jnp.full_like(max_scr, -jnp.inf)
        cnt_scr[...] = jnp.zeros_like(cnt_scr)

    x = x_ref[...]
    b = batch_ref[...]
    oh = (b == lax.broadcasted_iota(jnp.int32, (1, g), 1)).astype(F32)
    add_scr[...] += jnp.dot(oh.T, x, preferred_element_type=F32)
    cnt_scr[...] += jnp.dot(oh.T, jnp.ones_like(x), preferred_element_type=F32)
    for gg in range(g):
        cand = jnp.max(jnp.where(b == gg, x, -jnp.inf), axis=0, keepdims=True)
        max_scr[pl.ds(gg, 1), :] = jnp.maximum(max_scr[pl.ds(gg, 1), :], cand)

    @pl.when(i == nblocks - 1)
    def _final():
        cnt = cnt_scr[...]
        addp = add_scr[...]
        meanp = addp / jnp.maximum(cnt, 1.0)
        maxp = jnp.where(cnt > 0.0, max_scr[...], 0.0)
        o = jnp.concatenate([addp, meanp, maxp, u_ref[...]], axis=1)
        o = jnp.maximum(jnp.dot(o, w0_ref[...], preferred_element_type=F32) + b0_ref[...], 0.0)
        o = jnp.maximum(jnp.dot(o, w1_ref[...], preferred_element_type=F32) + b1_ref[...], 0.0)
        o = jnp.maximum(jnp.dot(o, w2_ref[...], preferred_element_type=F32) + b2_ref[...], 0.0)
        out_ref[...] = jnp.dot(o, w3_ref[...], preferred_element_type=F32) + b3_ref[...]


def _pool_mlp(x, batch2d, u, out_params, block=400):
    n, hid = x.shape
    g, udim = u.shape
    dim_out = out_params[3]["w"].shape[1]
    nblocks = n // block
    vec = lambda v: v.reshape(1, -1)
    cst = lambda shape: pl.BlockSpec(shape, lambda i: tuple(0 for _ in shape))
    return pl.pallas_call(
        functools.partial(_pool_body, nblocks),
        grid=(nblocks,),
        in_specs=[
            pl.BlockSpec((block, hid), lambda i: (i, 0)),
            pl.BlockSpec((block, 1), lambda i: (i, 0)),
            cst((g, udim)),
            cst((3 * hid + udim, hid)), cst((1, hid)),
            cst((hid, hid)), cst((1, hid)),
            cst((hid, hid)), cst((1, hid)),
            cst((hid, dim_out)), cst((1, dim_out)),
        ],
        out_specs=pl.BlockSpec((g, dim_out), lambda i: (0, 0)),
        out_shape=jax.ShapeDtypeStruct((g, dim_out), F32),
        scratch_shapes=[
            pltpu.VMEM((g, hid), F32),
            pltpu.VMEM((g, hid), F32),
            pltpu.VMEM((g, hid), F32),
        ],
    )(x, batch2d, u,
      out_params[0]["w"], vec(out_params[0]["b"]),
      out_params[1]["w"], vec(out_params[1]["b"]),
      out_params[2]["w"], vec(out_params[2]["b"]),
      out_params[3]["w"], vec(out_params[3]["b"]))


_NC, _NS, _LANES = 2, 16, 16
_NW = _NC * _NS


def _sc_mesh():
    return plsc.VectorSubcoreMesh(core_axis_name="c", subcore_axis_name="s")


def _wid():
    return lax.axis_index("s") * _NC + lax.axis_index("c")


def _gather_rows(tables, index_lists):
    ntab = len(tables)
    e = index_lists[0].shape[0]
    hid = tables[0].shape[1]
    nslices = _NW // ntab
    used = nslices * ntab
    epw = e // nslices
    ch = 80
    nch = epw // ch
    nslots = 4
    assert epw % ch == 0 and e % nslices == 0 and nch >= nslots

    scratch = (
        [pltpu.VMEM((epw,), jnp.int32)]
        + [pltpu.VMEM((ch, hid), F32) for _ in range(nslots)]
        + [pltpu.SemaphoreType.DMA for _ in range(nslots)]
        + [pltpu.SemaphoreType.DMA for _ in range(nslots)]
    )

    def body(*refs):
        tabs = refs[:ntab]
        idxs_hbm = refs[ntab:2 * ntab]
        outs = refs[2 * ntab:3 * ntab]
        sc = refs[3 * ntab:]
        idx_v = sc[0]
        bufs = sc[1:1 + nslots]
        gsem = sc[1 + nslots:1 + 2 * nslots]
        ssem = sc[1 + 2 * nslots:1 + 3 * nslots]
        w = _wid()

        for t in range(ntab):
            sl = w // ntab
            base = sl * epw

            @pl.when(w % ntab == t)
            def _(t=t, base=base):
                @pl.when(w < used)
                def _():
                    pltpu.sync_copy(idxs_hbm[t].at[pl.ds(base, epw)], idx_v)

                    def fire_gather(k, b):
                        pltpu.async_copy(
                            tabs[t].at[idx_v.at[pl.ds(k * ch, ch)]],
                            bufs[b], gsem[b])

                    def wait_gather(b):
                        pltpu.make_async_copy(
                            tabs[t].at[idx_v.at[pl.ds(0, ch)]],
                            bufs[b], gsem[b]).wait()

                    def fire_store(k, b):
                        pltpu.async_copy(
                            bufs[b], outs[t].at[pl.ds(base + k * ch, ch)],
                            ssem[b])

                    def wait_store(b):
                        pltpu.make_async_copy(
                            bufs[b], outs[t].at[pl.ds(base, ch)],
                            ssem[b]).wait()

                    for b in range(nslots - 1):
                        fire_gather(b, b)

                    def step(k4, carry):
                        for b in range(nslots):
                            k = k4 * nslots + b

                            @pl.when(k < nch)
                            def _(k=k, b=b):
                                wait_gather(b)
                                fire_store(k, b)
                                kn = k + nslots - 1

                                @pl.when(kn < nch)
                                def _():
                                    @pl.when(k >= 1)
                                    def _():
                                        wait_store((b + nslots - 1) % nslots)
                                    fire_gather(kn, (b + nslots - 1) % nslots)
                        return carry

                    lax.fori_loop(0, (nch + nslots - 1) // nslots, step, 0)
                    for i in range(nslots):
                        wait_store((nch - nslots + i) % nslots)

    out_type = [jax.ShapeDtypeStruct((e, hid), F32) for _ in range(ntab)]
    fn = pl.kernel(body, out_type=out_type, mesh=_sc_mesh(),
                   scratch_types=scratch)
    res = fn(*tables, *index_lists)
    return list(res) if ntab > 1 else [res]


def _segment_reduce(new_e, col_s, starts_pairs, n_pad):
    e, hid = new_e.shape
    rpt = n_pad // _NW
    ce = 96
    nf = hid // _LANES

    scratch = (
        [pltpu.VMEM((rpt + 1, hid), F32),
         pltpu.VMEM((rpt + 1, hid), F32),
         pltpu.VMEM(((rpt + 1) * _LANES,), F32),
         pltpu.VMEM((_LANES,), jnp.int32)]
        + [pltpu.VMEM((ce, hid), F32) for _ in range(2)]
        + [pltpu.VMEM((ce,), jnp.int32) for _ in range(2)]
        + [pltpu.SemaphoreType.DMA for _ in range(2)]
    )

    def body(vals_hbm, col_hbm, sp_hbm, s_out, mx_out, cnt_out,
             sumtbl, maxtbl, cnttbl, sbuf, vb0, vb1, cb0, cb1, sem0, sem1):
        vb = (vb0, vb1)
        cb = (cb0, cb1)
        sem = (sem0, sem1)
        w = _wid()
        node_base = w * rpt
        pltpu.sync_copy(sp_hbm.at[pl.ds(w * _LANES, _LANES)], sbuf)
        sv = sbuf[pl.ds(0, _LANES)]
        start = sv[0]
        end = sv[1]
        start8 = (start // 8) * 8
        nch = lax.div(end - start8 + (ce - 1), ce)

        zeros = jnp.zeros((_LANES,), F32)
        ninf = jnp.full((_LANES,), -jnp.inf, F32)

        def init_row(r, carry):
            for f in range(nf):
                sl = pl.ds(f * _LANES, _LANES)
                sumtbl[r, sl] = zeros
                maxtbl[r, sl] = ninf
            cnttbl[pl.ds(r * _LANES, _LANES)] = zeros
            return carry

        lax.fori_loop(0, rpt + 1, init_row, 0)

        emax = e - ce

        def fire(k, b):
            eb = jnp.minimum(start8 + k * ce, emax)
            pltpu.async_copy(col_hbm.at[pl.ds(eb, ce)], cb[b], sem[b])
            pltpu.async_copy(vals_hbm.at[pl.ds(eb, ce)], vb[b], sem[b])

        def wait(b):
            pltpu.make_async_copy(col_hbm.at[pl.ds(0, ce)], cb[b], sem[b]).wait()
            pltpu.make_async_copy(vals_hbm.at[pl.ds(0, ce)], vb[b], sem[b]).wait()

        @pl.when(nch > 0)
        def _():
            fire(0, 0)

        def flush(cur, cnt, accs, accm):
            @pl.when(cur >= 0)
            def _():
                for f in range(nf):
                    sl = pl.ds(f * _LANES, _LANES)
                    sumtbl[cur, sl] = sumtbl[cur, sl] + accs[f]
                    maxtbl[cur, sl] = jnp.maximum(maxtbl[cur, sl], accm[f])
                csl = pl.ds(cur * _LANES, _LANES)
                cnttbl[csl] = cnttbl[csl] + cnt

        def process(k, b):
            g0 = start8 + k * ce
            eb = jnp.minimum(g0, emax)
            lo = jnp.maximum(start, g0) - eb
            hi = jnp.minimum(end, g0 + ce) - eb
            zero_v = jnp.zeros((_LANES,), F32)
            ninf_v = jnp.full((_LANES,), -jnp.inf, F32)
            carry0 = ((jnp.int32(-1), jnp.float32(0.0))
                      + tuple(zero_v for _ in range(nf))
                      + tuple(ninf_v for _ in range(nf)))

            def group(gi, carry):
                base = gi * _LANES
                cvec = cb[b][pl.ds(base, _LANES)] - node_base
                for j in range(_LANES):
                    ei = base + j
                    active = (ei >= lo) & (ei < hi)
                    c = jnp.where(active, cvec[j], rpt)
                    vlist = [vb[b][ei, pl.ds(f * _LANES, _LANES)]
                             for f in range(nf)]
                    cur, cnt = carry[0], carry[1]
                    accs = carry[2:2 + nf]
                    accm = carry[2 + nf:]
                    is_new = c != cur

                    @pl.when(is_new)
                    def _():
                        flush(cur, cnt, accs, accm)

                    carry = ((c, jnp.where(is_new, 1.0, cnt + 1.0))
                             + tuple(jnp.where(is_new, v, a + v)
                                     for a, v in zip(accs, vlist))
                             + tuple(jnp.where(is_new, v, jnp.maximum(m, v))
                                     for m, v in zip(accm, vlist)))
                return carry

            fin = lax.fori_loop(0, ce // _LANES, group, carry0)
            flush(fin[0], fin[1], fin[2:2 + nf], fin[2 + nf:])

        def step(k2, carry):
            for b in (0, 1):
                k = k2 * 2 + b

                @pl.when(k < nch)
                def _():
                    wait(b)

                    @pl.when(k + 1 < nch)
                    def _():
                        fire(k + 1, 1 - b)

                    process(k, b)
            return carry

        lax.fori_loop(0, (nch + 1) // 2, step, 0)

        pltpu.sync_copy(sumtbl.at[pl.ds(0, rpt)], s_out.at[pl.ds(node_base, rpt)])
        pltpu.sync_copy(maxtbl.at[pl.ds(0, rpt)], mx_out.at[pl.ds(node_base, rpt)])
        pltpu.sync_copy(cnttbl.at[pl.ds(0, rpt * _LANES)],
                        cnt_out.at[pl.ds(node_base * _LANES, rpt * _LANES)])

    out_type = [
        jax.ShapeDtypeStruct((n_pad, hid), F32),
        jax.ShapeDtypeStruct((n_pad, hid), F32),
        jax.ShapeDtypeStruct((n_pad * _LANES,), F32),
    ]
    fn = pl.kernel(body, out_type=out_type, mesh=_sc_mesh(),
                   scratch_types=scratch)
    return fn(new_e, col_s, starts_pairs)


def kernel(x, edge_index, edge_attr, u, batch, params):
    n, f_node = x.shape
    e = edge_index.shape[1]
    hid = params["layers"][0]["e2"]["w"].shape[1]
    row, col = edge_index[0], edge_index[1]

    n_tiles = 32
    rows_per_tile = ((math.ceil(n / n_tiles) + 7) // 8) * 8
    n_pad = rows_per_tile * n_tiles
    perm = jnp.argsort(col).astype(jnp.int32)
    row_s = row[perm]
    col_s = col[perm]
    bounds = jnp.arange(n_tiles + 1, dtype=jnp.int32) * rows_per_tile
    starts = jnp.searchsorted(col_s, bounds, side="left").astype(jnp.int32)
    starts_pairs = jnp.pad(
        jnp.stack([starts[:-1], starts[1:]], axis=1), ((0, 0), (0, 14))).reshape(-1)

    batch2d = batch.reshape(n, 1)

    e_s = None
    for i, lp in enumerate(params["layers"]):
        residual = i > 0
        w1 = lp["e1"]["w"]
        wa, wb, wc = w1[:f_node], w1[f_node:2 * f_node], w1[2 * f_node:]
        xa, xb = _node_projections(x, wa, wb)
        if i == 0:
            ec0 = _matmul(edge_attr, wc)
            ga, gb, gc = _gather_rows([xa, xb, ec0], [row_s, col_s, perm])
            new_e = _edge_mlp_l0(ga, gb, gc, lp["e1"]["b"], lp["e2"]["w"], lp["e2"]["b"])
        else:
            ga, gb = _gather_rows([xa, xb], [row_s, col_s])
            new_e = _edge_mlp(ga, gb, e_s, wc, lp["e1"]["b"], lp["e2"]["w"], lp["e2"]["b"])
        e_s = new_e
        s, mx, cnt_flat = _segment_reduce(new_e, col_s, starts_pairs, n_pad)
        s, mx, cnt16 = s[:n], mx[:n], cnt_flat.reshape(n_pad, 16)[:n]
        x = _node_mlp(x, s, mx, cnt16, batch2d, u,
                      lp["n1"]["w"], lp["n1"]["b"], lp["n2"]["w"], lp["n2"]["b"],
                      residual)
    return _pool_mlp(x, batch2d, u, params["out"])


# speedup vs baseline: 2.1241x; 1.1006x over previous
import functools
import math

import jax
import jax.numpy as jnp
from jax import lax
from jax.experimental import pallas as pl
from jax.experimental.pallas import tpu as pltpu
from jax.experimental.pallas import tpu_sc as plsc

F32 = jnp.float32


def _prep_body(x_ref, wab_ref, xa_ref, xb_ref):
    y = jnp.dot(x_ref[...], wab_ref[...], preferred_element_type=F32)
    h = y.shape[1] // 2
    xa_ref[...] = y[:, :h]
    xb_ref[...] = y[:, h:]


def _node_projections(x, wa, wb, block=400):
    n, f = x.shape
    hid = wa.shape[1]
    wab = jnp.concatenate([wa, wb], axis=1)
    grid = (n // block,)
    return pl.pallas_call(
        _prep_body,
        grid=grid,
        in_specs=[
            pl.BlockSpec((block, f), lambda i: (i, 0)),
            pl.BlockSpec((f, 2 * hid), lambda i: (0, 0)),
        ],
        out_specs=[
            pl.BlockSpec((block, hid), lambda i: (i, 0)),
            pl.BlockSpec((block, hid), lambda i: (i, 0)),
        ],
        out_shape=[
            jax.ShapeDtypeStruct((n, hid), F32),
            jax.ShapeDtypeStruct((n, hid), F32),
        ],
    )(x, wab)


def _matmul_body(x_ref, w_ref, out_ref):
    out_ref[...] = jnp.dot(x_ref[...], w_ref[...], preferred_element_type=F32)


def _matmul(x, w, block=512):
    r, k = x.shape
    f = w.shape[1]
    return pl.pallas_call(
        _matmul_body,
        grid=(r // block,),
        in_specs=[
            pl.BlockSpec((block, k), lambda i: (i, 0)),
            pl.BlockSpec((k, f), lambda i: (0, 0)),
        ],
        out_specs=pl.BlockSpec((block, f), lambda i: (i, 0)),
        out_shape=jax.ShapeDtypeStruct((r, f), F32),
    )(x, w)


def _edge0_body(ga_ref, gb_ref, eas_ref, wc_ref, b1_ref, w2_ref, b2_ref, out_ref):
    ec = jnp.dot(eas_ref[...], wc_ref[...], preferred_element_type=F32)
    h = jnp.maximum(ga_ref[...] + gb_ref[...] + ec + b1_ref[...], 0.0)
    out_ref[...] = jnp.dot(h, w2_ref[...], preferred_element_type=F32) + b2_ref[...]


def _edge_body(ga_ref, gb_ref, es_ref, wc_ref, b1_ref, w2_ref, b2_ref, out_ref):
    es = es_ref[...]
    ec = jnp.dot(es, wc_ref[...], preferred_element_type=F32)
    h = jnp.maximum(ga_ref[...] + gb_ref[...] + ec + b1_ref[...], 0.0)
    out_ref[...] = jnp.dot(h, w2_ref[...], preferred_element_type=F32) + b2_ref[...] + es


def _edge_mlp_l0(ga, gb, eas, wcp, b1, w2, b2, block=512):
    e, hid = ga.shape
    fp = eas.shape[1]
    grid = (e // block,)
    vec = lambda v: v.reshape(1, -1)
    return pl.pallas_call(
        _edge0_body,
        grid=grid,
        in_specs=[
            pl.BlockSpec((block, hid), lambda i: (i, 0)),
            pl.BlockSpec((block, hid), lambda i: (i, 0)),
            pl.BlockSpec((block, fp), lambda i: (i, 0)),
            pl.BlockSpec((fp, hid), lambda i: (0, 0)),
            pl.BlockSpec((1, hid), lambda i: (0, 0)),
            pl.BlockSpec((hid, hid), lambda i: (0, 0)),
            pl.BlockSpec((1, hid), lambda i: (0, 0)),
        ],
        out_specs=pl.BlockSpec((block, hid), lambda i: (i, 0)),
        out_shape=jax.ShapeDtypeStruct((e, hid), F32),
    )(ga, gb, eas, wcp, vec(b1), w2, vec(b2))


def _edge_mlp(ga, gb, es, wc, b1, w2, b2, block=512):
    e, hid = ga.shape
    grid = (e // block,)
    vec = lambda v: v.reshape(1, -1)
    return pl.pallas_call(
        _edge_body,
        grid=grid,
        in_specs=[
            pl.BlockSpec((block, hid), lambda i: (i, 0)),
            pl.BlockSpec((block, hid), lambda i: (i, 0)),
            pl.BlockSpec((block, hid), lambda i: (i, 0)),
            pl.BlockSpec((hid, hid), lambda i: (0, 0)),
            pl.BlockSpec((1, hid), lambda i: (0, 0)),
            pl.BlockSpec((hid, hid), lambda i: (0, 0)),
            pl.BlockSpec((1, hid), lambda i: (0, 0)),
        ],
        out_specs=pl.BlockSpec((block, hid), lambda i: (i, 0)),
        out_shape=jax.ShapeDtypeStruct((e, hid), F32),
    )(ga, gb, es, wc, vec(b1), w2, vec(b2))


def _node_body(residual, x_ref, s_ref, mx_ref, cnt_ref, batch_ref, u_ref,
               w1_ref, b1_ref, w2_ref, b2_ref, out_ref):
    x = x_ref[...]
    s = s_ref[...]
    cnt = cnt_ref[...][:, :1]
    has = cnt > 0.0
    mx = jnp.where(has, mx_ref[...], 0.0)
    mean = s / jnp.maximum(cnt, 1.0)
    g = u_ref.shape[0]
    oh = (batch_ref[...] == lax.broadcasted_iota(jnp.int32, (1, g), 1)).astype(F32)
    ub = jnp.dot(oh, u_ref[...], preferred_element_type=F32)
    cat = jnp.concatenate([x, s, mx, mean, ub], axis=1)
    h = jnp.maximum(jnp.dot(cat, w1_ref[...], preferred_element_type=F32) + b1_ref[...], 0.0)
    o = jnp.dot(h, w2_ref[...], preferred_element_type=F32) + b2_ref[...]
    if residual:
        o = o + x
    out_ref[...] = o


def _node_mlp(x, s, mx, cnt, batch2d, u, w1, b1, w2, b2, residual, block=400):
    n, hid = x.shape
    g, udim = u.shape
    cin = w1.shape[0]
    grid = (n // block,)
    vec = lambda v: v.reshape(1, -1)
    return pl.pallas_call(
        functools.partial(_node_body, residual),
        grid=grid,
        in_specs=[
            pl.BlockSpec((block, hid), lambda i: (i, 0)),
            pl.BlockSpec((block, hid), lambda i: (i, 0)),
            pl.BlockSpec((block, hid), lambda i: (i, 0)),
            pl.BlockSpec((block, 16), lambda i: (i, 0)),
            pl.BlockSpec((block, 1), lambda i: (i, 0)),
            pl.BlockSpec((g, udim), lambda i: (0, 0)),
            pl.BlockSpec((cin, hid), lambda i: (0, 0)),
            pl.BlockSpec((1, hid), lambda i: (0, 0)),
            pl.BlockSpec((hid, hid), lambda i: (0, 0)),
            pl.BlockSpec((1, hid), lambda i: (0, 0)),
        ],
        out_specs=pl.BlockSpec((block, hid), lambda i: (i, 0)),
        out_shape=jax.ShapeDtypeStruct((n, hid), F32),
    )(x, s, mx, cnt, batch2d, u, w1, vec(b1), w2, vec(b2))


def _pool_body(nblocks, x_ref, batch_ref, u_ref,
               w0_ref, b0_ref, w1_ref, b1_ref, w2_ref, b2_ref, w3_ref, b3_ref,
               out_ref, add_scr, max_scr, cnt_scr):
    i = pl.program_id(0)
    g = u_ref.shape[0]

    @pl.when(i == 0)
    def _init():
        add_scr[...] = jnp.zeros_like(add_scr)
        max_scr[...] = jnp.full_like(max_scr, -jnp.inf)
        cnt_scr[...] = jnp.zeros_like(cnt_scr)

    x = x_ref[...]
    b = batch_ref[...]
    oh = (b == lax.broadcasted_iota(jnp.int32, (1, g), 1)).astype(F32)
    add_scr[...] += jnp.dot(oh.T, x, preferred_element_type=F32)
    cnt_scr[...] += jnp.dot(oh.T, jnp.ones_like(x), preferred_element_type=F32)
    for gg in range(g):
        cand = jnp.max(jnp.where(b == gg, x, -jnp.inf), axis=0, keepdims=True)
        max_scr[pl.ds(gg, 1), :] = jnp.maximum(max_scr[pl.ds(gg, 1), :], cand)

    @pl.when(i == nblocks - 1)
    def _final():
        cnt = cnt_scr[...]
        addp = add_scr[...]
        meanp = addp / jnp.maximum(cnt, 1.0)
        maxp = jnp.where(cnt > 0.0, max_scr[...], 0.0)
        o = jnp.concatenate([addp, meanp, maxp, u_ref[...]], axis=1)
        o = jnp.maximum(jnp.dot(o, w0_ref[...], preferred_element_type=F32) + b0_ref[...], 0.0)
        o = jnp.maximum(jnp.dot(o, w1_ref[...], preferred_element_type=F32) + b1_ref[...], 0.0)
        o = jnp.maximum(jnp.dot(o, w2_ref[...], preferred_element_type=F32) + b2_ref[...], 0.0)
        out_ref[...] = jnp.dot(o, w3_ref[...], preferred_element_type=F32) + b3_ref[...]


def _pool_mlp(x, batch2d, u, out_params, block=400):
    n, hid = x.shape
    g, udim = u.shape
    dim_out = out_params[3]["w"].shape[1]
    nblocks = n // block
    vec = lambda v: v.reshape(1, -1)
    cst = lambda shape: pl.BlockSpec(shape, lambda i: tuple(0 for _ in shape))
    return pl.pallas_call(
        functools.partial(_pool_body, nblocks),
        grid=(nblocks,),
        in_specs=[
            pl.BlockSpec((block, hid), lambda i: (i, 0)),
            pl.BlockSpec((block, 1), lambda i: (i, 0)),
            cst((g, udim)),
            cst((3 * hid + udim, hid)), cst((1, hid)),
            cst((hid, hid)), cst((1, hid)),
            cst((hid, hid)), cst((1, hid)),
            cst((hid, dim_out)), cst((1, dim_out)),
        ],
        out_specs=pl.BlockSpec((g, dim_out), lambda i: (0, 0)),
        out_shape=jax.ShapeDtypeStruct((g, dim_out), F32),
        scratch_shapes=[
            pltpu.VMEM((g, hid), F32),
            pltpu.VMEM((g, hid), F32),
            pltpu.VMEM((g, hid), F32),
        ],
    )(x, batch2d, u,
      out_params[0]["w"], vec(out_params[0]["b"]),
      out_params[1]["w"], vec(out_params[1]["b"]),
      out_params[2]["w"], vec(out_params[2]["b"]),
      out_params[3]["w"], vec(out_params[3]["b"]))


_NC, _NS, _LANES = 2, 16, 16
_NW = _NC * _NS


def _sc_mesh():
    return plsc.VectorSubcoreMesh(core_axis_name="c", subcore_axis_name="s")


def _wid():
    return lax.axis_index("s") * _NC + lax.axis_index("c")


def _gather_rows(tables, index_lists):
    ntab = len(tables)
    e = index_lists[0].shape[0]
    widths = [t.shape[1] for t in tables]
    nslices = _NW // ntab
    used = nslices * ntab
    epw = e // nslices
    ch = 80
    nch = epw // ch
    nslots = 4
    assert epw % ch == 0 and e % nslices == 0 and nch >= nslots

    scratch = (
        [pltpu.VMEM((epw,), jnp.int32)]
        + [pltpu.VMEM((ch, wd), F32)
           for wd in sorted(set(widths)) for _ in range(nslots)]
        + [pltpu.SemaphoreType.DMA for _ in range(nslots)]
        + [pltpu.SemaphoreType.DMA for _ in range(nslots)]
    )
    buf_off = {wd: 1 + i * nslots for i, wd in enumerate(sorted(set(widths)))}

    def body(*refs):
        tabs = refs[:ntab]
        idxs_hbm = refs[ntab:2 * ntab]
        outs = refs[2 * ntab:3 * ntab]
        sc = refs[3 * ntab:]
        idx_v = sc[0]
        nsem0 = 1 + len(set(widths)) * nslots
        gsem = sc[nsem0:nsem0 + nslots]
        ssem = sc[nsem0 + nslots:nsem0 + 2 * nslots]
        w = _wid()

        for t in range(ntab):
            sl = w // ntab
            base = sl * epw
            bufs = sc[buf_off[widths[t]]:buf_off[widths[t]] + nslots]

            @pl.when(w % ntab == t)
            def _(t=t, base=base, bufs=bufs):
                @pl.when(w < used)
                def _():
                    pltpu.sync_copy(idxs_hbm[t].at[pl.ds(base, epw)], idx_v)

                    def fire_gather(k, b):
                        pltpu.async_copy(
                            tabs[t].at[idx_v.at[pl.ds(k * ch, ch)]],
                            bufs[b], gsem[b])

                    def wait_gather(b):
                        pltpu.make_async_copy(
                            tabs[t].at[idx_v.at[pl.ds(0, ch)]],
                            bufs[b], gsem[b]).wait()

                    def fire_store(k, b):
                        pltpu.async_copy(
                            bufs[b], outs[t].at[pl.ds(base + k * ch, ch)],
                            ssem[b])

                    def wait_store(b):
                        pltpu.make_async_copy(
                            bufs[b], outs[t].at[pl.ds(base, ch)],
                            ssem[b]).wait()

                    for b in range(nslots - 1):
                        fire_gather(b, b)

                    def step(k4, carry):
                        for b in range(nslots):
                            k = k4 * nslots + b

                            @pl.when(k < nch)
                            def _(k=k, b=b):
                                wait_gather(b)
                                fire_store(k, b)
                                kn = k + nslots - 1

                                @pl.when(kn < nch)
                                def _():
                                    @pl.when(k >= 1)
                                    def _():
                                        wait_store((b + nslots - 1) % nslots)
                                    fire_gather(kn, (b + nslots - 1) % nslots)
                        return carry

                    lax.fori_loop(0, (nch + nslots - 1) // nslots, step, 0)
                    for i in range(nslots):
                        wait_store((nch - nslots + i) % nslots)

    out_type = [jax.ShapeDtypeStruct((e, wd), F32) for wd in widths]
    fn = pl.kernel(body, out_type=out_type, mesh=_sc_mesh(),
                   scratch_types=scratch)
    res = fn(*tables, *index_lists)
    return list(res) if ntab > 1 else [res]


def _segment_reduce(new_e, col_s, starts_pairs, n_pad):
    e, hid = new_e.shape
    rpt = n_pad // _NW
    ce = 96
    nf = hid // _LANES

    scratch = (
        [pltpu.VMEM((rpt + 1, hid), F32),
         pltpu.VMEM((rpt + 1, hid), F32),
         pltpu.VMEM(((rpt + 1) * _LANES,), F32),
         pltpu.VMEM((_LANES,), jnp.int32)]
        + [pltpu.VMEM((ce, hid), F32) for _ in range(2)]
        + [pltpu.VMEM((ce,), jnp.int32) for _ in range(2)]
        + [pltpu.SemaphoreType.DMA for _ in range(2)]
    )

    def body(vals_hbm, col_hbm, sp_hbm, s_out, mx_out, cnt_out,
             sumtbl, maxtbl, cnttbl, sbuf, vb0, vb1, cb0, cb1, sem0, sem1):
        vb = (vb0, vb1)
        cb = (cb0, cb1)
        sem = (sem0, sem1)
        w = _wid()
        node_base = w * rpt
        pltpu.sync_copy(sp_hbm.at[pl.ds(w * _LANES, _LANES)], sbuf)
        sv = sbuf[pl.ds(0, _LANES)]
        start = sv[0]
        end = sv[1]
        start8 = (start // 8) * 8
        nch = lax.div(end - start8 + (ce - 1), ce)

        zeros = jnp.zeros((_LANES,), F32)
        ninf = jnp.full((_LANES,), -jnp.inf, F32)

        def init_row(r, carry):
            for f in range(nf):
                sl = pl.ds(f * _LANES, _LANES)
                sumtbl[r, sl] = zeros
                maxtbl[r, sl] = ninf
            cnttbl[pl.ds(r * _LANES, _LANES)] = zeros
            return carry

        lax.fori_loop(0, rpt + 1, init_row, 0)

        emax = e - ce

        def fire(k, b):
            eb = jnp.minimum(start8 + k * ce, emax)
            pltpu.async_copy(col_hbm.at[pl.ds(eb, ce)], cb[b], sem[b])
            pltpu.async_copy(vals_hbm.at[pl.ds(eb, ce)], vb[b], sem[b])

        def wait(b):
            pltpu.make_async_copy(col_hbm.at[pl.ds(0, ce)], cb[b], sem[b]).wait()
            pltpu.make_async_copy(vals_hbm.at[pl.ds(0, ce)], vb[b], sem[b]).wait()

        @pl.when(nch > 0)
        def _():
            fire(0, 0)

        def flush(cur, cnt, accs, accm):
            @pl.when(cur >= 0)
            def _():
                for f in range(nf):
                    sl = pl.ds(f * _LANES, _LANES)
                    sumtbl[cur, sl] = sumtbl[cur, sl] + accs[f]
                    maxtbl[cur, sl] = jnp.maximum(maxtbl[cur, sl], accm[f])
                csl = pl.ds(cur * _LANES, _LANES)
                cnttbl[csl] = cnttbl[csl] + cnt

        def process(k, b):
            g0 = start8 + k * ce
            eb = jnp.minimum(g0, emax)
            lo = jnp.maximum(start, g0) - eb
            hi = jnp.minimum(end, g0 + ce) - eb
            zero_v = jnp.zeros((_LANES,), F32)
            ninf_v = jnp.full((_LANES,), -jnp.inf, F32)
            carry0 = ((jnp.int32(-1), jnp.float32(0.0))
                      + tuple(zero_v for _ in range(nf))
                      + tuple(ninf_v for _ in range(nf)))

            def group(gi, carry):
                base = gi * _LANES
                cvec = cb[b][pl.ds(base, _LANES)] - node_base
                for j in range(_LANES):
                    ei = base + j
                    active = (ei >= lo) & (ei < hi)
                    c = jnp.where(active, cvec[j], rpt)
                    vlist = [vb[b][ei, pl.ds(f * _LANES, _LANES)]
                             for f in range(nf)]
                    cur, cnt = carry[0], carry[1]
                    accs = carry[2:2 + nf]
                    accm = carry[2 + nf:]
                    is_new = c != cur

                    @pl.when(is_new)
                    def _():
                        flush(cur, cnt, accs, accm)

                    carry = ((c, jnp.where(is_new, 1.0, cnt + 1.0))
                             + tuple(jnp.where(is_new, v, a + v)
                                     for a, v in zip(accs, vlist))
                             + tuple(jnp.where(is_new, v, jnp.maximum(m, v))
                                     for m, v in zip(accm, vlist)))
                return carry

            fin = lax.fori_loop(0, ce // _LANES, group, carry0)
            flush(fin[0], fin[1], fin[2:2 + nf], fin[2 + nf:])

        def step(k2, carry):
            for b in (0, 1):
                k = k2 * 2 + b

                @pl.when(k < nch)
                def _():
                    wait(b)

                    @pl.when(k + 1 < nch)
                    def _():
                        fire(k + 1, 1 - b)

                    process(k, b)
            return carry

        lax.fori_loop(0, (nch + 1) // 2, step, 0)

        pltpu.sync_copy(sumtbl.at[pl.ds(0, rpt)], s_out.at[pl.ds(node_base, rpt)])
        pltpu.sync_copy(maxtbl.at[pl.ds(0, rpt)], mx_out.at[pl.ds(node_base, rpt)])
        pltpu.sync_copy(cnttbl.at[pl.ds(0, rpt * _LANES)],
                        cnt_out.at[pl.ds(node_base * _LANES, rpt * _LANES)])

    out_type = [
        jax.ShapeDtypeStruct((n_pad, hid), F32),
        jax.ShapeDtypeStruct((n_pad, hid), F32),
        jax.ShapeDtypeStruct((n_pad * _LANES,), F32),
    ]
    fn = pl.kernel(body, out_type=out_type, mesh=_sc_mesh(),
                   scratch_types=scratch)
    return fn(new_e, col_s, starts_pairs)


def kernel(x, edge_index, edge_attr, u, batch, params):
    n, f_node = x.shape
    e = edge_index.shape[1]
    hid = params["layers"][0]["e2"]["w"].shape[1]
    row, col = edge_index[0], edge_index[1]

    n_tiles = 32
    rows_per_tile = ((math.ceil(n / n_tiles) + 7) // 8) * 8
    n_pad = rows_per_tile * n_tiles
    f_edge = edge_attr.shape[1]
    ea_cols = [edge_attr[:, j] for j in range(f_edge)]
    srt = lax.sort((col, row, *ea_cols), num_keys=1)
    col_s, row_s = srt[0], srt[1]
    ea_sorted = jnp.stack(srt[2:], axis=1)
    bounds = jnp.arange(n_tiles + 1, dtype=jnp.int32) * rows_per_tile
    starts = jnp.searchsorted(col_s, bounds, side="left").astype(jnp.int32)
    starts_pairs = jnp.pad(
        jnp.stack([starts[:-1], starts[1:]], axis=1), ((0, 0), (0, 14))).reshape(-1)

    batch2d = batch.reshape(n, 1)

    e_s = None
    for i, lp in enumerate(params["layers"]):
        residual = i > 0
        w1 = lp["e1"]["w"]
        wa, wb, wc = w1[:f_node], w1[f_node:2 * f_node], w1[2 * f_node:]
        xa, xb = _node_projections(x, wa, wb)
        if i == 0:
            ga, gb = _gather_rows([xa, xb], [row_s, col_s])
            new_e = _edge_mlp_l0(ga, gb, ea_sorted, wc, lp["e1"]["b"], lp["e2"]["w"], lp["e2"]["b"])
        else:
            ga, gb = _gather_rows([xa, xb], [row_s, col_s])
            new_e = _edge_mlp(ga, gb, e_s, wc, lp["e1"]["b"], lp["e2"]["w"], lp["e2"]["b"])
        e_s = new_e
        s, mx, cnt_flat = _segment_reduce(new_e, col_s, starts_pairs, n_pad)
        s, mx, cnt16 = s[:n], mx[:n], cnt_flat.reshape(n_pad, 16)[:n]
        x = _node_mlp(x, s, mx, cnt16, batch2d, u,
                      lp["n1"]["w"], lp["n1"]["b"], lp["n2"]["w"], lp["n2"]["b"],
                      residual)
    return _pool_mlp(x, batch2d, u, params["out"])


# speedup vs baseline: 2.1911x; 1.0315x over previous
import functools
import math

import jax
import jax.numpy as jnp
from jax import lax
from jax.experimental import pallas as pl
from jax.experimental.pallas import tpu as pltpu
from jax.experimental.pallas import tpu_sc as plsc

F32 = jnp.float32


def _prep_body(x_ref, wab_ref, xa_ref, xb_ref):
    y = jnp.dot(x_ref[...], wab_ref[...], preferred_element_type=F32)
    h = y.shape[1] // 2
    xa_ref[...] = y[:, :h]
    xb_ref[...] = y[:, h:]


def _node_projections(x, wa, wb, block=400):
    n, f = x.shape
    hid = wa.shape[1]
    wab = jnp.concatenate([wa, wb], axis=1)
    grid = (n // block,)
    return pl.pallas_call(
        _prep_body,
        grid=grid,
        in_specs=[
            pl.BlockSpec((block, f), lambda i: (i, 0)),
            pl.BlockSpec((f, 2 * hid), lambda i: (0, 0)),
        ],
        out_specs=[
            pl.BlockSpec((block, hid), lambda i: (i, 0)),
            pl.BlockSpec((block, hid), lambda i: (i, 0)),
        ],
        out_shape=[
            jax.ShapeDtypeStruct((n, hid), F32),
            jax.ShapeDtypeStruct((n, hid), F32),
        ],
    )(x, wab)


def _matmul_body(x_ref, w_ref, out_ref):
    out_ref[...] = jnp.dot(x_ref[...], w_ref[...], preferred_element_type=F32)


def _matmul(x, w, block=512):
    r, k = x.shape
    f = w.shape[1]
    return pl.pallas_call(
        _matmul_body,
        grid=(r // block,),
        in_specs=[
            pl.BlockSpec((block, k), lambda i: (i, 0)),
            pl.BlockSpec((k, f), lambda i: (0, 0)),
        ],
        out_specs=pl.BlockSpec((block, f), lambda i: (i, 0)),
        out_shape=jax.ShapeDtypeStruct((r, f), F32),
    )(x, w)


def _edge0_body(ga_ref, gb_ref, eas_ref, wc_ref, b1_ref, w2_ref, b2_ref, out_ref):
    ec = jnp.dot(eas_ref[...], wc_ref[...], preferred_element_type=F32)
    h = jnp.maximum(ga_ref[...] + gb_ref[...] + ec + b1_ref[...], 0.0)
    out_ref[...] = jnp.dot(h, w2_ref[...], preferred_element_type=F32) + b2_ref[...]


def _edge_body(ga_ref, gb_ref, es_ref, wc_ref, b1_ref, w2_ref, b2_ref, out_ref):
    es = es_ref[...]
    ec = jnp.dot(es, wc_ref[...], preferred_element_type=F32)
    h = jnp.maximum(ga_ref[...] + gb_ref[...] + ec + b1_ref[...], 0.0)
    out_ref[...] = jnp.dot(h, w2_ref[...], preferred_element_type=F32) + b2_ref[...] + es


def _edge_mlp_l0(ga, gb, eas, wcp, b1, w2, b2, block=512):
    e, hid = ga.shape
    fp = eas.shape[1]
    grid = (e // block,)
    vec = lambda v: v.reshape(1, -1)
    return pl.pallas_call(
        _edge0_body,
        grid=grid,
        in_specs=[
            pl.BlockSpec((block, hid), lambda i: (i, 0)),
            pl.BlockSpec((block, hid), lambda i: (i, 0)),
            pl.BlockSpec((block, fp), lambda i: (i, 0)),
            pl.BlockSpec((fp, hid), lambda i: (0, 0)),
            pl.BlockSpec((1, hid), lambda i: (0, 0)),
            pl.BlockSpec((hid, hid), lambda i: (0, 0)),
            pl.BlockSpec((1, hid), lambda i: (0, 0)),
        ],
        out_specs=pl.BlockSpec((block, hid), lambda i: (i, 0)),
        out_shape=jax.ShapeDtypeStruct((e, hid), F32),
    )(ga, gb, eas, wcp, vec(b1), w2, vec(b2))


def _edge_mlp(ga, gb, es, wc, b1, w2, b2, block=512):
    e, hid = ga.shape
    grid = (e // block,)
    vec = lambda v: v.reshape(1, -1)
    return pl.pallas_call(
        _edge_body,
        grid=grid,
        in_specs=[
            pl.BlockSpec((block, hid), lambda i: (i, 0)),
            pl.BlockSpec((block, hid), lambda i: (i, 0)),
            pl.BlockSpec((block, hid), lambda i: (i, 0)),
            pl.BlockSpec((hid, hid), lambda i: (0, 0)),
            pl.BlockSpec((1, hid), lambda i: (0, 0)),
            pl.BlockSpec((hid, hid), lambda i: (0, 0)),
            pl.BlockSpec((1, hid), lambda i: (0, 0)),
        ],
        out_specs=pl.BlockSpec((block, hid), lambda i: (i, 0)),
        out_shape=jax.ShapeDtypeStruct((e, hid), F32),
    )(ga, gb, es, wc, vec(b1), w2, vec(b2))


def _node_body(residual, x_ref, s_ref, mx_ref, cnt_ref, batch_ref, u_ref,
               w1_ref, b1_ref, w2_ref, b2_ref, out_ref):
    x = x_ref[...]
    s = s_ref[...]
    cnt = cnt_ref[...][:, :1]
    has = cnt > 0.0
    mx = jnp.where(has, mx_ref[...], 0.0)
    mean = s / jnp.maximum(cnt, 1.0)
    g = u_ref.shape[0]
    oh = (batch_ref[...] == lax.broadcasted_iota(jnp.int32, (1, g), 1)).astype(F32)
    ub = jnp.dot(oh, u_ref[...], preferred_element_type=F32)
    cat = jnp.concatenate([x, s, mx, mean, ub], axis=1)
    h = jnp.maximum(jnp.dot(cat, w1_ref[...], preferred_element_type=F32) + b1_ref[...], 0.0)
    o = jnp.dot(h, w2_ref[...], preferred_element_type=F32) + b2_ref[...]
    if residual:
        o = o + x
    out_ref[...] = o


def _node_mlp(x, s, mx, cnt, batch2d, u, w1, b1, w2, b2, residual, block=400):
    n, hid = x.shape
    g, udim = u.shape
    cin = w1.shape[0]
    grid = (n // block,)
    vec = lambda v: v.reshape(1, -1)
    return pl.pallas_call(
        functools.partial(_node_body, residual),
        grid=grid,
        in_specs=[
            pl.BlockSpec((block, hid), lambda i: (i, 0)),
            pl.BlockSpec((block, hid), lambda i: (i, 0)),
            pl.BlockSpec((block, hid), lambda i: (i, 0)),
            pl.BlockSpec((block, 16), lambda i: (i, 0)),
            pl.BlockSpec((block, 1), lambda i: (i, 0)),
            pl.BlockSpec((g, udim), lambda i: (0, 0)),
            pl.BlockSpec((cin, hid), lambda i: (0, 0)),
            pl.BlockSpec((1, hid), lambda i: (0, 0)),
            pl.BlockSpec((hid, hid), lambda i: (0, 0)),
            pl.BlockSpec((1, hid), lambda i: (0, 0)),
        ],
        out_specs=pl.BlockSpec((block, hid), lambda i: (i, 0)),
        out_shape=jax.ShapeDtypeStruct((n, hid), F32),
    )(x, s, mx, cnt, batch2d, u, w1, vec(b1), w2, vec(b2))


def _pool_body(nblocks, x_ref, batch_ref, u_ref,
               w0_ref, b0_ref, w1_ref, b1_ref, w2_ref, b2_ref, w3_ref, b3_ref,
               out_ref, add_scr, max_scr, cnt_scr):
    i = pl.program_id(0)
    g = u_ref.shape[0]

    @pl.when(i == 0)
    def _init():
        add_scr[...] = jnp.zeros_like(add_scr)
        max_scr[...] = jnp.full_like(max_scr, -jnp.inf)
        cnt_scr[...] = jnp.zeros_like(cnt_scr)

    x = x_ref[...]
    b = batch_ref[...]
    oh = (b == lax.broadcasted_iota(jnp.int32, (1, g), 1)).astype(F32)
    add_scr[...] += jnp.dot(oh.T, x, preferred_element_type=F32)
    cnt_scr[...] += jnp.dot(oh.T, jnp.ones_like(x), preferred_element_type=F32)
    for gg in range(g):
        cand = jnp.max(jnp.where(b == gg, x, -jnp.inf), axis=0, keepdims=True)
        max_scr[pl.ds(gg, 1), :] = jnp.maximum(max_scr[pl.ds(gg, 1), :], cand)

    @pl.when(i == nblocks - 1)
    def _final():
        cnt = cnt_scr[...]
        addp = add_scr[...]
        meanp = addp / jnp.maximum(cnt, 1.0)
        maxp = jnp.where(cnt > 0.0, max_scr[...], 0.0)
        o = jnp.concatenate([addp, meanp, maxp, u_ref[...]], axis=1)
        o = jnp.maximum(jnp.dot(o, w0_ref[...], preferred_element_type=F32) + b0_ref[...], 0.0)
        o = jnp.maximum(jnp.dot(o, w1_ref[...], preferred_element_type=F32) + b1_ref[...], 0.0)
        o = jnp.maximum(jnp.dot(o, w2_ref[...], preferred_element_type=F32) + b2_ref[...], 0.0)
        out_ref[...] = jnp.dot(o, w3_ref[...], preferred_element_type=F32) + b3_ref[...]


def _pool_mlp(x, batch2d, u, out_params, block=400):
    n, hid = x.shape
    g, udim = u.shape
    dim_out = out_params[3]["w"].shape[1]
    nblocks = n // block
    vec = lambda v: v.reshape(1, -1)
    cst = lambda shape: pl.BlockSpec(shape, lambda i: tuple(0 for _ in shape))
    return pl.pallas_call(
        functools.partial(_pool_body, nblocks),
        grid=(nblocks,),
        in_specs=[
            pl.BlockSpec((block, hid), lambda i: (i, 0)),
            pl.BlockSpec((block, 1), lambda i: (i, 0)),
            cst((g, udim)),
            cst((3 * hid + udim, hid)), cst((1, hid)),
            cst((hid, hid)), cst((1, hid)),
            cst((hid, hid)), cst((1, hid)),
            cst((hid, dim_out)), cst((1, dim_out)),
        ],
        out_specs=pl.BlockSpec((g, dim_out), lambda i: (0, 0)),
        out_shape=jax.ShapeDtypeStruct((g, dim_out), F32),
        scratch_shapes=[
            pltpu.VMEM((g, hid), F32),
            pltpu.VMEM((g, hid), F32),
            pltpu.VMEM((g, hid), F32),
        ],
    )(x, batch2d, u,
      out_params[0]["w"], vec(out_params[0]["b"]),
      out_params[1]["w"], vec(out_params[1]["b"]),
      out_params[2]["w"], vec(out_params[2]["b"]),
      out_params[3]["w"], vec(out_params[3]["b"]))


_NC, _NS, _LANES = 2, 16, 16
_NW = _NC * _NS


def _sc_mesh():
    return plsc.VectorSubcoreMesh(core_axis_name="c", subcore_axis_name="s")


def _wid():
    return lax.axis_index("s") * _NC + lax.axis_index("c")


def _gather_rows(tables, index_lists):
    ntab = len(tables)
    e = index_lists[0].shape[0]
    widths = [t.shape[1] for t in tables]
    nslices = _NW // ntab
    used = nslices * ntab
    epw = e // nslices
    ch = 80
    nch = epw // ch
    nslots = 4
    assert epw % ch == 0 and e % nslices == 0 and nch >= nslots

    scratch = (
        [pltpu.VMEM((epw,), jnp.int32)]
        + [pltpu.VMEM((ch, wd), F32)
           for wd in sorted(set(widths)) for _ in range(nslots)]
        + [pltpu.SemaphoreType.DMA for _ in range(nslots)]
        + [pltpu.SemaphoreType.DMA for _ in range(nslots)]
    )
    buf_off = {wd: 1 + i * nslots for i, wd in enumerate(sorted(set(widths)))}

    def body(*refs):
        tabs = refs[:ntab]
        idxs_hbm = refs[ntab:2 * ntab]
        outs = refs[2 * ntab:3 * ntab]
        sc = refs[3 * ntab:]
        idx_v = sc[0]
        nsem0 = 1 + len(set(widths)) * nslots
        gsem = sc[nsem0:nsem0 + nslots]
        ssem = sc[nsem0 + nslots:nsem0 + 2 * nslots]
        w = _wid()

        for t in range(ntab):
            sl = w // ntab
            base = sl * epw
            bufs = sc[buf_off[widths[t]]:buf_off[widths[t]] + nslots]

            @pl.when(w % ntab == t)
            def _(t=t, base=base, bufs=bufs):
                @pl.when(w < used)
                def _():
                    pltpu.sync_copy(idxs_hbm[t].at[pl.ds(base, epw)], idx_v)

                    def fire_gather(k, b):
                        pltpu.async_copy(
                            tabs[t].at[idx_v.at[pl.ds(k * ch, ch)]],
                            bufs[b], gsem[b])

                    def wait_gather(b):
                        pltpu.make_async_copy(
                            tabs[t].at[idx_v.at[pl.ds(0, ch)]],
                            bufs[b], gsem[b]).wait()

                    def fire_store(k, b):
                        pltpu.async_copy(
                            bufs[b], outs[t].at[pl.ds(base + k * ch, ch)],
                            ssem[b])

                    def wait_store(b):
                        pltpu.make_async_copy(
                            bufs[b], outs[t].at[pl.ds(base, ch)],
                            ssem[b]).wait()

                    for b in range(nslots - 1):
                        fire_gather(b, b)

                    def step(k4, carry):
                        for b in range(nslots):
                            k = k4 * nslots + b

                            @pl.when(k < nch)
                            def _(k=k, b=b):
                                wait_gather(b)
                                fire_store(k, b)
                                kn = k + nslots - 1

                                @pl.when(kn < nch)
                                def _():
                                    @pl.when(k >= 1)
                                    def _():
                                        wait_store((b + nslots - 1) % nslots)
                                    fire_gather(kn, (b + nslots - 1) % nslots)
                        return carry

                    lax.fori_loop(0, (nch + nslots - 1) // nslots, step, 0)
                    for i in range(nslots):
                        wait_store((nch - nslots + i) % nslots)

    out_type = [jax.ShapeDtypeStruct((e, wd), F32) for wd in widths]
    fn = pl.kernel(body, out_type=out_type, mesh=_sc_mesh(),
                   scratch_types=scratch)
    res = fn(*tables, *index_lists)
    return list(res) if ntab > 1 else [res]


def _segment_reduce(new_e, col_s, starts_pairs, n_pad):
    e, hid = new_e.shape
    rpt = n_pad // _NW
    ce = 96
    nf = hid // _LANES

    scratch = (
        [pltpu.VMEM((rpt + 1, hid), F32),
         pltpu.VMEM((rpt + 1, hid), F32),
         pltpu.VMEM(((rpt + 1) * _LANES,), F32),
         pltpu.VMEM((_LANES,), jnp.int32)]
        + [pltpu.VMEM((ce, hid), F32) for _ in range(2)]
        + [pltpu.VMEM((ce,), jnp.int32) for _ in range(2)]
        + [pltpu.SemaphoreType.DMA for _ in range(2)]
    )

    def body(vals_hbm, col_hbm, sp_hbm, s_out, mx_out, cnt_out,
             sumtbl, maxtbl, cnttbl, sbuf, vb0, vb1, cb0, cb1, sem0, sem1):
        vb = (vb0, vb1)
        cb = (cb0, cb1)
        sem = (sem0, sem1)
        w = _wid()
        node_base = w * rpt
        pltpu.sync_copy(sp_hbm.at[pl.ds(w * _LANES, _LANES)], sbuf)
        sv = sbuf[pl.ds(0, _LANES)]
        start = sv[0]
        end = sv[1]
        start8 = (start // 8) * 8
        nch = lax.div(end - start8 + (ce - 1), ce)

        zeros = jnp.zeros((_LANES,), F32)
        ninf = jnp.full((_LANES,), -jnp.inf, F32)

        def init_row(r, carry):
            for f in range(nf):
                sl = pl.ds(f * _LANES, _LANES)
                sumtbl[r, sl] = zeros
                maxtbl[r, sl] = ninf
            cnttbl[pl.ds(r * _LANES, _LANES)] = zeros
            return carry

        lax.fori_loop(0, rpt + 1, init_row, 0)

        emax = e - ce

        def fire(k, b):
            eb = jnp.minimum(start8 + k * ce, emax)
            pltpu.async_copy(col_hbm.at[pl.ds(eb, ce)], cb[b], sem[b])
            pltpu.async_copy(vals_hbm.at[pl.ds(eb, ce)], vb[b], sem[b])

        def wait(b):
            pltpu.make_async_copy(col_hbm.at[pl.ds(0, ce)], cb[b], sem[b]).wait()
            pltpu.make_async_copy(vals_hbm.at[pl.ds(0, ce)], vb[b], sem[b]).wait()

        @pl.when(nch > 0)
        def _():
            fire(0, 0)

        def flush(cur, cnt, accs, accm):
            @pl.when(cur >= 0)
            def _():
                for f in range(nf):
                    sl = pl.ds(f * _LANES, _LANES)
                    sumtbl[cur, sl] = sumtbl[cur, sl] + accs[f]
                    maxtbl[cur, sl] = jnp.maximum(maxtbl[cur, sl], accm[f])
                csl = pl.ds(cur * _LANES, _LANES)
                cnttbl[csl] = cnttbl[csl] + cnt

        def process(k, b):
            g0 = start8 + k * ce
            eb = jnp.minimum(g0, emax)
            lo = jnp.maximum(start, g0) - eb
            hi = jnp.minimum(end, g0 + ce) - eb
            zero_v = jnp.zeros((_LANES,), F32)
            ninf_v = jnp.full((_LANES,), -jnp.inf, F32)
            carry0 = ((jnp.int32(-1), jnp.float32(0.0))
                      + tuple(zero_v for _ in range(nf))
                      + tuple(ninf_v for _ in range(nf)))

            def group(gi, carry):
                base = gi * _LANES
                cvec = cb[b][pl.ds(base, _LANES)] - node_base
                for j in range(_LANES):
                    ei = base + j
                    active = (ei >= lo) & (ei < hi)
                    c = jnp.where(active, cvec[j], rpt)
                    vlist = [vb[b][ei, pl.ds(f * _LANES, _LANES)]
                             for f in range(nf)]
                    cur, cnt = carry[0], carry[1]
                    accs = carry[2:2 + nf]
                    accm = carry[2 + nf:]
                    is_new = c != cur

                    @pl.when(is_new)
                    def _():
                        flush(cur, cnt, accs, accm)

                    carry = ((c, jnp.where(is_new, 1.0, cnt + 1.0))
                             + tuple(jnp.where(is_new, v, a + v)
                                     for a, v in zip(accs, vlist))
                             + tuple(jnp.where(is_new, v, jnp.maximum(m, v))
                                     for m, v in zip(accm, vlist)))
                return carry

            fin = lax.fori_loop(0, ce // _LANES, group, carry0)
            flush(fin[0], fin[1], fin[2:2 + nf], fin[2 + nf:])

        def step(k2, carry):
            for b in (0, 1):
                k = k2 * 2 + b

                @pl.when(k < nch)
                def _():
                    wait(b)

                    @pl.when(k + 1 < nch)
                    def _():
                        fire(k + 1, 1 - b)

                    process(k, b)
            return carry

        lax.fori_loop(0, (nch + 1) // 2, step, 0)

        pltpu.sync_copy(sumtbl.at[pl.ds(0, rpt)], s_out.at[pl.ds(node_base, rpt)])
        pltpu.sync_copy(maxtbl.at[pl.ds(0, rpt)], mx_out.at[pl.ds(node_base, rpt)])
        pltpu.sync_copy(cnttbl.at[pl.ds(0, rpt * _LANES)],
                        cnt_out.at[pl.ds(node_base * _LANES, rpt * _LANES)])

    out_type = [
        jax.ShapeDtypeStruct((n_pad, hid), F32),
        jax.ShapeDtypeStruct((n_pad, hid), F32),
        jax.ShapeDtypeStruct((n_pad * _LANES,), F32),
    ]
    fn = pl.kernel(body, out_type=out_type, mesh=_sc_mesh(),
                   scratch_types=scratch)
    return fn(new_e, col_s, starts_pairs)


def kernel(x, edge_index, edge_attr, u, batch, params):
    n, f_node = x.shape
    e = edge_index.shape[1]
    hid = params["layers"][0]["e2"]["w"].shape[1]
    row, col = edge_index[0], edge_index[1]

    n_tiles = 32
    rows_per_tile = ((math.ceil(n / n_tiles) + 7) // 8) * 8
    n_pad = rows_per_tile * n_tiles
    eid = jnp.arange(e, dtype=jnp.int32)
    col_s, row_s, perm = lax.sort((col, row, eid), num_keys=1)
    ea_sorted = edge_attr[perm]
    bounds = jnp.arange(n_tiles + 1, dtype=jnp.int32) * rows_per_tile
    starts = jnp.searchsorted(col_s, bounds, side="left").astype(jnp.int32)
    starts_pairs = jnp.pad(
        jnp.stack([starts[:-1], starts[1:]], axis=1), ((0, 0), (0, 14))).reshape(-1)

    batch2d = batch.reshape(n, 1)

    e_s = None
    for i, lp in enumerate(params["layers"]):
        residual = i > 0
        w1 = lp["e1"]["w"]
        wa, wb, wc = w1[:f_node], w1[f_node:2 * f_node], w1[2 * f_node:]
        xa, xb = _node_projections(x, wa, wb)
        if i == 0:
            ga, gb = _gather_rows([xa, xb], [row_s, col_s])
            new_e = _edge_mlp_l0(ga, gb, ea_sorted, wc, lp["e1"]["b"], lp["e2"]["w"], lp["e2"]["b"])
        else:
            ga, gb = _gather_rows([xa, xb], [row_s, col_s])
            new_e = _edge_mlp(ga, gb, e_s, wc, lp["e1"]["b"], lp["e2"]["w"], lp["e2"]["b"])
        e_s = new_e
        s, mx, cnt_flat = _segment_reduce(new_e, col_s, starts_pairs, n_pad)
        s, mx, cnt16 = s[:n], mx[:n], cnt_flat.reshape(n_pad, 16)[:n]
        x = _node_mlp(x, s, mx, cnt16, batch2d, u,
                      lp["n1"]["w"], lp["n1"]["b"], lp["n2"]["w"], lp["n2"]["b"],
                      residual)
    return _pool_mlp(x, batch2d, u, params["out"])


# speedup vs baseline: 2.4704x; 1.1275x over previous
import functools
import math

import jax
import jax.numpy as jnp
from jax import lax
from jax.experimental import pallas as pl
from jax.experimental.pallas import tpu as pltpu
from jax.experimental.pallas import tpu_sc as plsc

F32 = jnp.float32


def _prep_body(x_ref, wab_ref, xa_ref, xb_ref):
    y = jnp.dot(x_ref[...], wab_ref[...], preferred_element_type=F32)
    h = y.shape[1] // 2
    xa_ref[...] = y[:, :h]
    xb_ref[...] = y[:, h:]


def _node_projections(x, wa, wb, block=400):
    n, f = x.shape
    hid = wa.shape[1]
    wab = jnp.concatenate([wa, wb], axis=1)
    grid = (n // block,)
    return pl.pallas_call(
        _prep_body,
        grid=grid,
        in_specs=[
            pl.BlockSpec((block, f), lambda i: (i, 0)),
            pl.BlockSpec((f, 2 * hid), lambda i: (0, 0)),
        ],
        out_specs=[
            pl.BlockSpec((block, hid), lambda i: (i, 0)),
            pl.BlockSpec((block, hid), lambda i: (i, 0)),
        ],
        out_shape=[
            jax.ShapeDtypeStruct((n, hid), F32),
            jax.ShapeDtypeStruct((n, hid), F32),
        ],
    )(x, wab)


def _matmul_body(x_ref, w_ref, out_ref):
    out_ref[...] = jnp.dot(x_ref[...], w_ref[...], preferred_element_type=F32)


def _matmul(x, w, block=512):
    r, k = x.shape
    f = w.shape[1]
    return pl.pallas_call(
        _matmul_body,
        grid=(r // block,),
        in_specs=[
            pl.BlockSpec((block, k), lambda i: (i, 0)),
            pl.BlockSpec((k, f), lambda i: (0, 0)),
        ],
        out_specs=pl.BlockSpec((block, f), lambda i: (i, 0)),
        out_shape=jax.ShapeDtypeStruct((r, f), F32),
    )(x, w)


def _edge0_body(g_ref, eas_ref, wc_ref, b1_ref, w2_ref, b2_ref, out_ref):
    ec = jnp.dot(eas_ref[...], wc_ref[...], preferred_element_type=F32)
    h = jnp.maximum(g_ref[...] + ec + b1_ref[...], 0.0)
    out_ref[...] = jnp.dot(h, w2_ref[...], preferred_element_type=F32) + b2_ref[...]


def _edge_body(g_ref, es_ref, wc_ref, b1_ref, w2_ref, b2_ref, out_ref):
    es = es_ref[...]
    ec = jnp.dot(es, wc_ref[...], preferred_element_type=F32)
    h = jnp.maximum(g_ref[...] + ec + b1_ref[...], 0.0)
    out_ref[...] = jnp.dot(h, w2_ref[...], preferred_element_type=F32) + b2_ref[...] + es


def _edge_mlp_l0(g, eas, wcp, b1, w2, b2, block=512):
    e, hid = g.shape
    fp = eas.shape[1]
    grid = (e // block,)
    vec = lambda v: v.reshape(1, -1)
    return pl.pallas_call(
        _edge0_body,
        grid=grid,
        in_specs=[
            pl.BlockSpec((block, hid), lambda i: (i, 0)),
            pl.BlockSpec((block, fp), lambda i: (i, 0)),
            pl.BlockSpec((fp, hid), lambda i: (0, 0)),
            pl.BlockSpec((1, hid), lambda i: (0, 0)),
            pl.BlockSpec((hid, hid), lambda i: (0, 0)),
            pl.BlockSpec((1, hid), lambda i: (0, 0)),
        ],
        out_specs=pl.BlockSpec((block, hid), lambda i: (i, 0)),
        out_shape=jax.ShapeDtypeStruct((e, hid), F32),
    )(g, eas, wcp, vec(b1), w2, vec(b2))


def _edge_mlp(g, es, wc, b1, w2, b2, block=512):
    e, hid = g.shape
    grid = (e // block,)
    vec = lambda v: v.reshape(1, -1)
    return pl.pallas_call(
        _edge_body,
        grid=grid,
        in_specs=[
            pl.BlockSpec((block, hid), lambda i: (i, 0)),
            pl.BlockSpec((block, hid), lambda i: (i, 0)),
            pl.BlockSpec((hid, hid), lambda i: (0, 0)),
            pl.BlockSpec((1, hid), lambda i: (0, 0)),
            pl.BlockSpec((hid, hid), lambda i: (0, 0)),
            pl.BlockSpec((1, hid), lambda i: (0, 0)),
        ],
        out_specs=pl.BlockSpec((block, hid), lambda i: (i, 0)),
        out_shape=jax.ShapeDtypeStruct((e, hid), F32),
    )(g, es, wc, vec(b1), w2, vec(b2))


def _node_body(residual, x_ref, s_ref, mx_ref, cnt_ref, batch_ref, u_ref,
               w1_ref, b1_ref, w2_ref, b2_ref, out_ref):
    x = x_ref[...]
    s = s_ref[...]
    cnt = cnt_ref[...][:, :1]
    has = cnt > 0.0
    mx = jnp.where(has, mx_ref[...], 0.0)
    mean = s / jnp.maximum(cnt, 1.0)
    g = u_ref.shape[0]
    oh = (batch_ref[...] == lax.broadcasted_iota(jnp.int32, (1, g), 1)).astype(F32)
    ub = jnp.dot(oh, u_ref[...], preferred_element_type=F32)
    cat = jnp.concatenate([x, s, mx, mean, ub], axis=1)
    h = jnp.maximum(jnp.dot(cat, w1_ref[...], preferred_element_type=F32) + b1_ref[...], 0.0)
    o = jnp.dot(h, w2_ref[...], preferred_element_type=F32) + b2_ref[...]
    if residual:
        o = o + x
    out_ref[...] = o


def _node_mlp(x, s, mx, cnt, batch2d, u, w1, b1, w2, b2, residual, block=400):
    n, hid = x.shape
    g, udim = u.shape
    cin = w1.shape[0]
    grid = (n // block,)
    vec = lambda v: v.reshape(1, -1)
    return pl.pallas_call(
        functools.partial(_node_body, residual),
        grid=grid,
        in_specs=[
            pl.BlockSpec((block, hid), lambda i: (i, 0)),
            pl.BlockSpec((block, hid), lambda i: (i, 0)),
            pl.BlockSpec((block, hid), lambda i: (i, 0)),
            pl.BlockSpec((block, 16), lambda i: (i, 0)),
            pl.BlockSpec((block, 1), lambda i: (i, 0)),
            pl.BlockSpec((g, udim), lambda i: (0, 0)),
            pl.BlockSpec((cin, hid), lambda i: (0, 0)),
            pl.BlockSpec((1, hid), lambda i: (0, 0)),
            pl.BlockSpec((hid, hid), lambda i: (0, 0)),
            pl.BlockSpec((1, hid), lambda i: (0, 0)),
        ],
        out_specs=pl.BlockSpec((block, hid), lambda i: (i, 0)),
        out_shape=jax.ShapeDtypeStruct((n, hid), F32),
    )(x, s, mx, cnt, batch2d, u, w1, vec(b1), w2, vec(b2))


def _pool_body(nblocks, x_ref, batch_ref, u_ref,
               w0_ref, b0_ref, w1_ref, b1_ref, w2_ref, b2_ref, w3_ref, b3_ref,
               out_ref, add_scr, max_scr, cnt_scr):
    i = pl.program_id(0)
    g = u_ref.shape[0]

    @pl.when(i == 0)
    def _init():
        add_scr[...] = jnp.zeros_like(add_scr)
        max_scr[...] = jnp.full_like(max_scr, -jnp.inf)
        cnt_scr[...] = jnp.zeros_like(cnt_scr)

    x = x_ref[...]
    b = batch_ref[...]
    oh = (b == lax.broadcasted_iota(jnp.int32, (1, g), 1)).astype(F32)
    add_scr[...] += jnp.dot(oh.T, x, preferred_element_type=F32)
    cnt_scr[...] += jnp.dot(oh.T, jnp.ones_like(x), preferred_element_type=F32)
    for gg in range(g):
        cand = jnp.max(jnp.where(b == gg, x, -jnp.inf), axis=0, keepdims=True)
        max_scr[pl.ds(gg, 1), :] = jnp.maximum(max_scr[pl.ds(gg, 1), :], cand)

    @pl.when(i == nblocks - 1)
    def _final():
        cnt = cnt_scr[...]
        addp = add_scr[...]
        meanp = addp / jnp.maximum(cnt, 1.0)
        maxp = jnp.where(cnt > 0.0, max_scr[...], 0.0)
        o = jnp.concatenate([addp, meanp, maxp, u_ref[...]], axis=1)
        o = jnp.maximum(jnp.dot(o, w0_ref[...], preferred_element_type=F32) + b0_ref[...], 0.0)
        o = jnp.maximum(jnp.dot(o, w1_ref[...], preferred_element_type=F32) + b1_ref[...], 0.0)
        o = jnp.maximum(jnp.dot(o, w2_ref[...], preferred_element_type=F32) + b2_ref[...], 0.0)
        out_ref[...] = jnp.dot(o, w3_ref[...], preferred_element_type=F32) + b3_ref[...]


def _pool_mlp(x, batch2d, u, out_params, block=400):
    n, hid = x.shape
    g, udim = u.shape
    dim_out = out_params[3]["w"].shape[1]
    nblocks = n // block
    vec = lambda v: v.reshape(1, -1)
    cst = lambda shape: pl.BlockSpec(shape, lambda i: tuple(0 for _ in shape))
    return pl.pallas_call(
        functools.partial(_pool_body, nblocks),
        grid=(nblocks,),
        in_specs=[
            pl.BlockSpec((block, hid), lambda i: (i, 0)),
            pl.BlockSpec((block, 1), lambda i: (i, 0)),
            cst((g, udim)),
            cst((3 * hid + udim, hid)), cst((1, hid)),
            cst((hid, hid)), cst((1, hid)),
            cst((hid, hid)), cst((1, hid)),
            cst((hid, dim_out)), cst((1, dim_out)),
        ],
        out_specs=pl.BlockSpec((g, dim_out), lambda i: (0, 0)),
        out_shape=jax.ShapeDtypeStruct((g, dim_out), F32),
        scratch_shapes=[
            pltpu.VMEM((g, hid), F32),
            pltpu.VMEM((g, hid), F32),
            pltpu.VMEM((g, hid), F32),
        ],
    )(x, batch2d, u,
      out_params[0]["w"], vec(out_params[0]["b"]),
      out_params[1]["w"], vec(out_params[1]["b"]),
      out_params[2]["w"], vec(out_params[2]["b"]),
      out_params[3]["w"], vec(out_params[3]["b"]))


_NC, _NS, _LANES = 2, 16, 16
_NW = _NC * _NS


def _sc_mesh():
    return plsc.VectorSubcoreMesh(core_axis_name="c", subcore_axis_name="s")


def _wid():
    return lax.axis_index("s") * _NC + lax.axis_index("c")


def _gather_add(xa, xb, rows_s, cols_s):
    e = rows_s.shape[0]
    hid = xa.shape[1]
    nf = hid // _LANES
    epw = e // _NW
    ch = 80
    nch = epw // ch
    ns = 3
    assert epw % ch == 0 and e % _NW == 0 and nch >= ns

    scratch = (
        [pltpu.VMEM((epw,), jnp.int32) for _ in range(2)]
        + [pltpu.VMEM((ch, hid), F32) for _ in range(2 * ns)]
        + [pltpu.SemaphoreType.DMA for _ in range(3 * ns)]
    )

    def body(xa_hbm, xb_hbm, rows_hbm, cols_hbm, g_out, idxa, idxb, *sc):
        bufa = sc[0:ns]
        bufb = sc[ns:2 * ns]
        gsa = sc[2 * ns:3 * ns]
        gsb = sc[3 * ns:4 * ns]
        ssem = sc[4 * ns:5 * ns]
        w = _wid()
        base = w * epw
        pltpu.sync_copy(rows_hbm.at[pl.ds(base, epw)], idxa)
        pltpu.sync_copy(cols_hbm.at[pl.ds(base, epw)], idxb)

        def fire_gather(k, b):
            pltpu.async_copy(xa_hbm.at[idxa.at[pl.ds(k * ch, ch)]], bufa[b], gsa[b])
            pltpu.async_copy(xb_hbm.at[idxb.at[pl.ds(k * ch, ch)]], bufb[b], gsb[b])

        def wait_gather(b):
            pltpu.make_async_copy(xa_hbm.at[idxa.at[pl.ds(0, ch)]], bufa[b], gsa[b]).wait()
            pltpu.make_async_copy(xb_hbm.at[idxb.at[pl.ds(0, ch)]], bufb[b], gsb[b]).wait()

        def fire_store(k, b):
            pltpu.async_copy(bufa[b], g_out.at[pl.ds(base + k * ch, ch)], ssem[b])

        def wait_store(b):
            pltpu.make_async_copy(bufa[b], g_out.at[pl.ds(base, ch)], ssem[b]).wait()

        def add_chunk(b):
            def rowfn(r, carry):
                for f in range(nf):
                    sl = pl.ds(f * _LANES, _LANES)
                    bufa[b][r, sl] = bufa[b][r, sl] + bufb[b][r, sl]
                return carry
            lax.fori_loop(0, ch, rowfn, 0)

        for b in range(ns - 1):
            fire_gather(b, b)

        def step(kq, carry):
            for b in range(ns):
                k = kq * ns + b

                @pl.when(k < nch)
                def _(k=k, b=b):
                    wait_gather(b)
                    kn = k + ns - 1

                    @pl.when(kn < nch)
                    def _():
                        @pl.when(k >= 1)
                        def _():
                            wait_store((b + ns - 1) % ns)
                        fire_gather(kn, (b + ns - 1) % ns)

                    add_chunk(b)
                    fire_store(k, b)
            return carry

        lax.fori_loop(0, (nch + ns - 1) // ns, step, 0)
        for i in range(ns):
            wait_store((nch - ns + i) % ns)

    fn = pl.kernel(body,
                   out_type=jax.ShapeDtypeStruct((e, hid), F32),
                   mesh=_sc_mesh(), scratch_types=scratch)
    return fn(xa, xb, rows_s, cols_s)


def _segment_reduce(new_e, col_s, starts_pairs, n_pad):
    e, hid = new_e.shape
    rpt = n_pad // _NW
    ce = 96
    nf = hid // _LANES

    scratch = (
        [pltpu.VMEM((rpt + 1, hid), F32),
         pltpu.VMEM((rpt + 1, hid), F32),
         pltpu.VMEM(((rpt + 1) * _LANES,), F32),
         pltpu.VMEM((_LANES,), jnp.int32)]
        + [pltpu.VMEM((ce, hid), F32) for _ in range(2)]
        + [pltpu.VMEM((ce,), jnp.int32) for _ in range(2)]
        + [pltpu.SemaphoreType.DMA for _ in range(2)]
    )

    def body(vals_hbm, col_hbm, sp_hbm, s_out, mx_out, cnt_out,
             sumtbl, maxtbl, cnttbl, sbuf, vb0, vb1, cb0, cb1, sem0, sem1):
        vb = (vb0, vb1)
        cb = (cb0, cb1)
        sem = (sem0, sem1)
        w = _wid()
        node_base = w * rpt
        pltpu.sync_copy(sp_hbm.at[pl.ds(w * _LANES, _LANES)], sbuf)
        sv = sbuf[pl.ds(0, _LANES)]
        start = sv[0]
        end = sv[1]
        start8 = (start // 8) * 8
        nch = lax.div(end - start8 + (ce - 1), ce)

        zeros = jnp.zeros((_LANES,), F32)
        ninf = jnp.full((_LANES,), -jnp.inf, F32)

        def init_row(r, carry):
            for f in range(nf):
                sl = pl.ds(f * _LANES, _LANES)
                sumtbl[r, sl] = zeros
                maxtbl[r, sl] = ninf
            cnttbl[pl.ds(r * _LANES, _LANES)] = zeros
            return carry

        lax.fori_loop(0, rpt + 1, init_row, 0)

        emax = e - ce

        def fire(k, b):
            eb = jnp.minimum(start8 + k * ce, emax)
            pltpu.async_copy(col_hbm.at[pl.ds(eb, ce)], cb[b], sem[b])
            pltpu.async_copy(vals_hbm.at[pl.ds(eb, ce)], vb[b], sem[b])

        def wait(b):
            pltpu.make_async_copy(col_hbm.at[pl.ds(0, ce)], cb[b], sem[b]).wait()
            pltpu.make_async_copy(vals_hbm.at[pl.ds(0, ce)], vb[b], sem[b]).wait()

        @pl.when(nch > 0)
        def _():
            fire(0, 0)

        def flush(cur, cnt, accs, accm):
            @pl.when(cur >= 0)
            def _():
                for f in range(nf):
                    sl = pl.ds(f * _LANES, _LANES)
                    sumtbl[cur, sl] = sumtbl[cur, sl] + accs[f]
                    maxtbl[cur, sl] = jnp.maximum(maxtbl[cur, sl], accm[f])
                csl = pl.ds(cur * _LANES, _LANES)
                cnttbl[csl] = cnttbl[csl] + cnt

        def process(k, b):
            g0 = start8 + k * ce
            eb = jnp.minimum(g0, emax)
            lo = jnp.maximum(start, g0) - eb
            hi = jnp.minimum(end, g0 + ce) - eb
            zero_v = jnp.zeros((_LANES,), F32)
            ninf_v = jnp.full((_LANES,), -jnp.inf, F32)
            carry0 = ((jnp.int32(-1), jnp.float32(0.0))
                      + tuple(zero_v for _ in range(nf))
                      + tuple(ninf_v for _ in range(nf)))

            def group(gi, carry):
                base = gi * _LANES
                cvec = cb[b][pl.ds(base, _LANES)] - node_base
                for j in range(_LANES):
                    ei = base + j
                    active = (ei >= lo) & (ei < hi)
                    c = jnp.where(active, cvec[j], rpt)
                    vlist = [vb[b][ei, pl.ds(f * _LANES, _LANES)]
                             for f in range(nf)]
                    cur, cnt = carry[0], carry[1]
                    accs = carry[2:2 + nf]
                    accm = carry[2 + nf:]
                    is_new = c != cur

                    @pl.when(is_new)
                    def _():
                        flush(cur, cnt, accs, accm)

                    carry = ((c, jnp.where(is_new, 1.0, cnt + 1.0))
                             + tuple(jnp.where(is_new, v, a + v)
                                     for a, v in zip(accs, vlist))
                             + tuple(jnp.where(is_new, v, jnp.maximum(m, v))
                                     for m, v in zip(accm, vlist)))
                return carry

            fin = lax.fori_loop(0, ce // _LANES, group, carry0)
            flush(fin[0], fin[1], fin[2:2 + nf], fin[2 + nf:])

        def step(k2, carry):
            for b in (0, 1):
                k = k2 * 2 + b

                @pl.when(k < nch)
                def _():
                    wait(b)

                    @pl.when(k + 1 < nch)
                    def _():
                        fire(k + 1, 1 - b)

                    process(k, b)
            return carry

        lax.fori_loop(0, (nch + 1) // 2, step, 0)

        pltpu.sync_copy(sumtbl.at[pl.ds(0, rpt)], s_out.at[pl.ds(node_base, rpt)])
        pltpu.sync_copy(maxtbl.at[pl.ds(0, rpt)], mx_out.at[pl.ds(node_base, rpt)])
        pltpu.sync_copy(cnttbl.at[pl.ds(0, rpt * _LANES)],
                        cnt_out.at[pl.ds(node_base * _LANES, rpt * _LANES)])

    out_type = [
        jax.ShapeDtypeStruct((n_pad, hid), F32),
        jax.ShapeDtypeStruct((n_pad, hid), F32),
        jax.ShapeDtypeStruct((n_pad * _LANES,), F32),
    ]
    fn = pl.kernel(body, out_type=out_type, mesh=_sc_mesh(),
                   scratch_types=scratch)
    return fn(new_e, col_s, starts_pairs)


def kernel(x, edge_index, edge_attr, u, batch, params):
    n, f_node = x.shape
    e = edge_index.shape[1]
    hid = params["layers"][0]["e2"]["w"].shape[1]
    row, col = edge_index[0], edge_index[1]

    n_tiles = 32
    rows_per_tile = ((math.ceil(n / n_tiles) + 7) // 8) * 8
    n_pad = rows_per_tile * n_tiles
    eid = jnp.arange(e, dtype=jnp.int32)
    col_s, row_s, perm = lax.sort((col, row, eid), num_keys=1)
    ea_sorted = edge_attr[perm]
    bounds = jnp.arange(n_tiles + 1, dtype=jnp.int32) * rows_per_tile
    starts = jnp.searchsorted(col_s, bounds, side="left").astype(jnp.int32)
    starts_pairs = jnp.pad(
        jnp.stack([starts[:-1], starts[1:]], axis=1), ((0, 0), (0, 14))).reshape(-1)

    batch2d = batch.reshape(n, 1)

    e_s = None
    for i, lp in enumerate(params["layers"]):
        residual = i > 0
        w1 = lp["e1"]["w"]
        wa, wb, wc = w1[:f_node], w1[f_node:2 * f_node], w1[2 * f_node:]
        xa, xb = _node_projections(x, wa, wb)
        g = _gather_add(xa, xb, row_s, col_s)
        if i == 0:
            new_e = _edge_mlp_l0(g, ea_sorted, wc, lp["e1"]["b"], lp["e2"]["w"], lp["e2"]["b"])
        else:
            new_e = _edge_mlp(g, e_s, wc, lp["e1"]["b"], lp["e2"]["w"], lp["e2"]["b"])
        e_s = new_e
        s, mx, cnt_flat = _segment_reduce(new_e, col_s, starts_pairs, n_pad)
        s, mx, cnt16 = s[:n], mx[:n], cnt_flat.reshape(n_pad, 16)[:n]
        x = _node_mlp(x, s, mx, cnt16, batch2d, u,
                      lp["n1"]["w"], lp["n1"]["b"], lp["n2"]["w"], lp["n2"]["b"],
                      residual)
    return _pool_mlp(x, batch2d, u, params["out"])
